# Initial kernel scaffold; baseline (speedup 1.0000x reference)
import jax
import jax.numpy as jnp
from jax.experimental import pallas as pl


def kernel(x, edge_index, W1, b1, W2, b2, W3, b3):
    raise NotImplementedError("write your pallas kernel here")


# speedup vs baseline: 21.8352x; 21.8352x over previous
import functools

import jax
import jax.numpy as jnp
from jax import lax
from jax.experimental import pallas as pl
from jax.experimental.pallas import tpu as pltpu
from jax.experimental.pallas import tpu_sc as plsc

N = 10000
NP = 10240
E = 320000
NW = 32
C = 128
EP = 327680
CHE = EP // (NW * C)
CHC = EP // (16 * C)
RPT = NP // 16
BR = 1024
G = NP // BR

_MESH = plsc.VectorSubcoreMesh(core_axis_name="c", subcore_axis_name="s")


def _make_deg_kernel():
    scratch = [
        pltpu.VMEM((CHE, C), jnp.int32),
        pltpu.VMEM((C, 16), jnp.float32),
        pltpu.VMEM_SHARED((NP, 16), jnp.float32),
    ]

    @functools.partial(
        pl.kernel, mesh=_MESH,
        out_type=jax.ShapeDtypeStruct((2, NP, 16), jnp.float32),
        scratch_types=scratch,
        compiler_params=pltpu.CompilerParams(use_tc_tiling_on_sc=False))
    def deg_kernel(dsts_hbm, ones_hbm, zeros_hbm, out_hbm, didx, ones_v, acc):
        c = lax.axis_index("c")
        s = lax.axis_index("s")
        w = c * 16 + s
        pltpu.sync_copy(dsts_hbm.at[w], didx)
        pltpu.sync_copy(ones_hbm, ones_v)
        pltpu.sync_copy(zeros_hbm.at[pl.ds(s * RPT, RPT)],
                        acc.at[pl.ds(s * RPT, RPT)])
        plsc.subcore_barrier()
        for j in range(CHE):
            pltpu.sync_copy(ones_v, acc.at[didx.at[j]], add=True)
        plsc.subcore_barrier()
        pltpu.sync_copy(acc.at[pl.ds(s * RPT, RPT)],
                        out_hbm.at[c].at[pl.ds(s * RPT, RPT)])

    return deg_kernel


def _make_col_agg_kernel():
    scratch = [
        pltpu.VMEM((CHC, C), jnp.int32),
        pltpu.VMEM((CHC, C), jnp.int32),
        pltpu.VMEM((C, 64), jnp.float32),
        pltpu.VMEM((C, 64), jnp.float32),
        pltpu.VMEM_SHARED((NP, 64), jnp.float32),
        pltpu.SemaphoreType.DMA,
        pltpu.SemaphoreType.DMA,
    ]

    @functools.partial(
        pl.kernel, mesh=_MESH,
        out_type=jax.ShapeDtypeStruct((2, NP, 64), jnp.float32),
        scratch_types=scratch,
        compiler_params=pltpu.CompilerParams(use_tc_tiling_on_sc=False))
    def col_kernel(srcs_hbm, dsts_hbm, zeros_hbm, table, out_hbm,
                   sidx, didx, rows0, rows1, acc, sem0, sem1):
        bufs = (rows0, rows1)
        sems = (sem0, sem1)
        c = lax.axis_index("c")
        s = lax.axis_index("s")
        pltpu.sync_copy(srcs_hbm.at[c * 16 + s], sidx)
        pltpu.sync_copy(dsts_hbm.at[s], didx)
        pltpu.sync_copy(zeros_hbm.at[pl.ds(s * RPT, RPT)],
                        acc.at[pl.ds(s * RPT, RPT)])
        plsc.subcore_barrier()
        cps = [None, None]
        cps[0] = pltpu.async_copy(table.at[sidx.at[0]], bufs[0], sems[0])
        for j in range(CHC):
            if j + 1 < CHC:
                cps[(j + 1) % 2] = pltpu.async_copy(
                    table.at[sidx.at[j + 1]], bufs[(j + 1) % 2],
                    sems[(j + 1) % 2])
            cps[j % 2].wait()
            pltpu.sync_copy(bufs[j % 2], acc.at[didx.at[j]], add=True)
        plsc.subcore_barrier()
        pltpu.sync_copy(acc.at[pl.ds(s * RPT, RPT)],
                        out_hbm.at[c].at[pl.ds(s * RPT, RPT)])

    return col_kernel


def _make_l3_kernel():
    scratch = [
        pltpu.VMEM((CHE, C), jnp.int32),
        pltpu.VMEM((CHE, C), jnp.int32),
        pltpu.VMEM((C, 48), jnp.float32),
        pltpu.VMEM((C, 48), jnp.float32),
        pltpu.VMEM_SHARED((NP, 48), jnp.float32),
        pltpu.SemaphoreType.DMA,
        pltpu.SemaphoreType.DMA,
    ]

    @functools.partial(
        pl.kernel, mesh=_MESH,
        out_type=jax.ShapeDtypeStruct((2, NP, 48), jnp.float32),
        scratch_types=scratch,
        compiler_params=pltpu.CompilerParams(use_tc_tiling_on_sc=False))
    def l3_kernel(srcs_hbm, dsts_hbm, zeros_hbm, table, out_hbm,
                  sidx, didx, rows0, rows1, acc, sem0, sem1):
        bufs = (rows0, rows1)
        sems = (sem0, sem1)
        c = lax.axis_index("c")
        s = lax.axis_index("s")
        w = c * 16 + s
        pltpu.sync_copy(srcs_hbm.at[w], sidx)
        pltpu.sync_copy(dsts_hbm.at[w], didx)
        pltpu.sync_copy(zeros_hbm.at[pl.ds(s * RPT, RPT)],
                        acc.at[pl.ds(s * RPT, RPT)])
        plsc.subcore_barrier()
        cps = [None, None]
        cps[0] = pltpu.async_copy(table.at[sidx.at[0]], bufs[0], sems[0])
        for j in range(CHE):
            if j + 1 < CHE:
                cps[(j + 1) % 2] = pltpu.async_copy(
                    table.at[sidx.at[j + 1]], bufs[(j + 1) % 2],
                    sems[(j + 1) % 2])
            cps[j % 2].wait()
            pltpu.sync_copy(bufs[j % 2], acc.at[didx.at[j]], add=True)
        plsc.subcore_barrier()
        pltpu.sync_copy(acc.at[pl.ds(s * RPT, RPT)],
                        out_hbm.at[c].at[pl.ds(s * RPT, RPT)])

    return l3_kernel


def _dot(a, b):
    return lax.dot_general(a, b, (((1,), (0,)), ((), ())),
                           precision=lax.Precision.HIGHEST,
                           preferred_element_type=jnp.float32)


def _row_spec(width):
    return pl.BlockSpec((BR, width), lambda i: (i, 0))


def _half_spec():
    return pl.BlockSpec((2, BR, 64), lambda i: (0, i, 0))


def _part_spec(width):
    return pl.BlockSpec((2, BR, width), lambda i: (0, i, 0))


def _full_spec(shape):
    ndim = len(shape)
    return pl.BlockSpec(shape, lambda i, _n=ndim: (0,) * _n)


def _split2(h):
    return jnp.stack([h[:, :64], h[:, 64:128]], axis=0)


def _tc0_body(pref, xref, dinv_out, xp_out):
    deg = pref[0] + pref[1] + 1.0
    dinv = lax.rsqrt(deg)
    dinv_out[...] = dinv
    xp_out[...] = _split2(xref[...] * dinv[:, 0:1])


def _tc1_body(pref, xpref, dref, w1ref, b1ref, ta_out, tb_out):
    dinv = dref[...][:, 0:1]
    agg = jnp.concatenate([pref[0] + xpref[0], pref[1] + xpref[1]], axis=1)
    y = agg * dinv
    h = jnp.maximum(_dot(y, w1ref[...]) + b1ref[...], 0.0)
    hp = h * dinv
    ta_out[...] = _split2(hp[:, :128])
    tb_out[...] = _split2(hp[:, 128:])


def _tc2_body(paref, pbref, taref, tbref, dref, w2ref, b2ref, w3ref, h3_out):
    dinv = dref[...][:, 0:1]
    agg = jnp.concatenate([paref[0] + taref[0], paref[1] + taref[1],
                           pbref[0] + tbref[0], pbref[1] + tbref[1]], axis=1)
    y = agg * dinv
    h2 = jnp.maximum(_dot(y, w2ref[...]) + b2ref[...], 0.0)
    h3_out[...] = _dot(h2, w3ref[...]) * dinv


def _tc3_body(pref, h3ref, dref, b3ref, out):
    dinv = dref[...][:, 0:1]
    y = (pref[0] + pref[1] + h3ref[...]) * dinv + b3ref[...]
    mask = lax.broadcasted_iota(jnp.int32, (BR, 48), 1) < 40
    ym = jnp.where(mask, y, -jnp.inf)
    mx = jnp.max(ym, axis=1, keepdims=True)
    e = jnp.where(mask, jnp.exp(y - mx), 0.0)
    out[...] = y - mx - jnp.log(jnp.sum(e, axis=1, keepdims=True))


def kernel(x, edge_index, W1, b1, W2, b2, W3, b3):
    f32 = jnp.float32
    src = edge_index[0].astype(jnp.int32)
    dst = edge_index[1].astype(jnp.int32)
    pad_idx = N + (jnp.arange(EP - E, dtype=jnp.int32) % (NP - N))
    src_p = jnp.concatenate([src, pad_idx])
    dst_p = jnp.concatenate([dst, pad_idx])
    srcs_e = src_p.reshape(NW, CHE, C)
    dsts_e = dst_p.reshape(NW, CHE, C)
    src_c = src_p.reshape(16, CHC, C)
    srcs_c = jnp.concatenate([src_c, src_c + NP], axis=0)
    dsts_c = dst_p.reshape(16, CHC, C)

    x_pad = jnp.pad(x.astype(f32), ((0, NP - N), (0, 0)))
    ones16 = jnp.ones((C, 16), f32)
    zeros16 = jnp.zeros((NP, 16), f32)
    zeros48 = jnp.zeros((NP, 48), f32)
    zeros64 = jnp.zeros((NP, 64), f32)
    w3p = jnp.pad(W3.astype(f32), ((0, 0), (0, 8)))
    b1r = b1.astype(f32).reshape(1, 256)
    b2r = b2.astype(f32).reshape(1, 256)
    b3r = jnp.pad(b3.astype(f32), (0, 8)).reshape(1, 48)

    col_agg = _make_col_agg_kernel()

    deg_parts = _make_deg_kernel()(dsts_e, ones16, zeros16)

    dinv, xp2 = pl.pallas_call(
        _tc0_body,
        grid=(G,),
        in_specs=[_part_spec(16), _row_spec(128)],
        out_specs=[_row_spec(16), _half_spec()],
        out_shape=[jax.ShapeDtypeStruct((NP, 16), f32),
                   jax.ShapeDtypeStruct((2, NP, 64), f32)],
    )(deg_parts, x_pad)

    p1 = col_agg(srcs_c, dsts_c, zeros64, xp2.reshape(2 * NP, 64))

    ta, tb = pl.pallas_call(
        _tc1_body,
        grid=(G,),
        in_specs=[_half_spec(), _half_spec(), _row_spec(16),
                  _full_spec((128, 256)), _full_spec((1, 256))],
        out_specs=[_half_spec(), _half_spec()],
        out_shape=[jax.ShapeDtypeStruct((2, NP, 64), f32),
                   jax.ShapeDtypeStruct((2, NP, 64), f32)],
    )(p1, xp2, dinv, W1.astype(f32), b1r)

    pa = col_agg(srcs_c, dsts_c, zeros64, ta.reshape(2 * NP, 64))
    pb = col_agg(srcs_c, dsts_c, zeros64, tb.reshape(2 * NP, 64))

    (h3,) = pl.pallas_call(
        _tc2_body,
        grid=(G,),
        in_specs=[_half_spec(), _half_spec(), _half_spec(), _half_spec(),
                  _row_spec(16), _full_spec((256, 256)),
                  _full_spec((1, 256)), _full_spec((256, 48))],
        out_specs=[_row_spec(48)],
        out_shape=[jax.ShapeDtypeStruct((NP, 48), f32)],
    )(pa, pb, ta, tb, dinv, W2.astype(f32), b2r, w3p)

    p3 = _make_l3_kernel()(srcs_e, dsts_e, zeros48, h3)

    (out48,) = pl.pallas_call(
        _tc3_body,
        grid=(G,),
        in_specs=[_part_spec(48), _row_spec(48), _row_spec(16),
                  _full_spec((1, 48))],
        out_specs=[_row_spec(48)],
        out_shape=[jax.ShapeDtypeStruct((NP, 48), f32)],
    )(p3, h3, dinv, b3r)

    return out48[:N, :40]


# speedup vs baseline: 24.7703x; 1.1344x over previous
import functools

import jax
import jax.numpy as jnp
from jax import lax
from jax.experimental import pallas as pl
from jax.experimental.pallas import tpu as pltpu
from jax.experimental.pallas import tpu_sc as plsc

N = 10000
NP = 10240
E = 320000
NW = 32
C = 256
EP = 327680
CHE = EP // (NW * C)
CHC = EP // (16 * C)
RPT = NP // 16
BR = 1024
G = NP // BR

_MESH = plsc.VectorSubcoreMesh(core_axis_name="c", subcore_axis_name="s")
_NOTC = pltpu.CompilerParams(use_tc_tiling_on_sc=False)


def _make_deg_kernel():
    scratch = [
        pltpu.VMEM((CHE, C), jnp.int32),
        pltpu.VMEM((C, 16), jnp.float32),
        pltpu.VMEM_SHARED((NP, 16), jnp.float32),
    ]

    @functools.partial(
        pl.kernel, mesh=_MESH,
        out_type=jax.ShapeDtypeStruct((2, NP, 16), jnp.float32),
        scratch_types=scratch, compiler_params=_NOTC)
    def deg_kernel(dsts_hbm, ones_hbm, zeros_hbm, out_hbm, didx, ones_v, acc):
        c = lax.axis_index("c")
        s = lax.axis_index("s")
        w = c * 16 + s
        pltpu.sync_copy(dsts_hbm.at[w], didx)
        pltpu.sync_copy(ones_hbm, ones_v)
        pltpu.sync_copy(zeros_hbm.at[pl.ds(s * RPT, RPT)],
                        acc.at[pl.ds(s * RPT, RPT)])
        plsc.subcore_barrier()
        for j in range(CHE):
            pltpu.sync_copy(ones_v, acc.at[didx.at[j]], add=True)
        plsc.subcore_barrier()
        pltpu.sync_copy(acc.at[pl.ds(s * RPT, RPT)],
                        out_hbm.at[c].at[pl.ds(s * RPT, RPT)])

    return deg_kernel


def _agg_body(nch, table, out_hbm, sidx, didx, rows0, rows1, acc, sem0, sem1,
              c, s):
    bufs = (rows0, rows1)
    sems = (sem0, sem1)
    plsc.subcore_barrier()
    cps = [None, None]
    cps[0] = pltpu.async_copy(table.at[sidx.at[0]], bufs[0], sems[0])
    for j in range(nch):
        if j + 1 < nch:
            cps[(j + 1) % 2] = pltpu.async_copy(
                table.at[sidx.at[j + 1]], bufs[(j + 1) % 2],
                sems[(j + 1) % 2])
        cps[j % 2].wait()
        pltpu.sync_copy(bufs[j % 2], acc.at[didx.at[j]], add=True)
    plsc.subcore_barrier()
    pltpu.sync_copy(acc.at[pl.ds(s * RPT, RPT)],
                    out_hbm.at[c].at[pl.ds(s * RPT, RPT)])


def _make_col_agg_kernel():
    scratch = [
        pltpu.VMEM((CHC, C), jnp.int32),
        pltpu.VMEM((CHC, C), jnp.int32),
        pltpu.VMEM((C, 64), jnp.float32),
        pltpu.VMEM((C, 64), jnp.float32),
        pltpu.VMEM_SHARED((NP, 64), jnp.float32),
        pltpu.SemaphoreType.DMA,
        pltpu.SemaphoreType.DMA,
    ]

    @functools.partial(
        pl.kernel, mesh=_MESH,
        out_type=jax.ShapeDtypeStruct((2, NP, 64), jnp.float32),
        scratch_types=scratch, compiler_params=_NOTC)
    def col_kernel(srcs_hbm, dsts_hbm, zeros_hbm, table, out_hbm,
                   sidx, didx, rows0, rows1, acc, sem0, sem1):
        c = lax.axis_index("c")
        s = lax.axis_index("s")
        pltpu.sync_copy(srcs_hbm.at[c * 16 + s], sidx)
        pltpu.sync_copy(dsts_hbm.at[s], didx)
        pltpu.sync_copy(zeros_hbm.at[pl.ds(s * RPT, RPT)],
                        acc.at[pl.ds(s * RPT, RPT)])
        _agg_body(CHC, table, out_hbm, sidx, didx, rows0, rows1, acc,
                  sem0, sem1, c, s)

    return col_kernel


def _make_l3_kernel():
    scratch = [
        pltpu.VMEM((CHE, C), jnp.int32),
        pltpu.VMEM((CHE, C), jnp.int32),
        pltpu.VMEM((C, 48), jnp.float32),
        pltpu.VMEM((C, 48), jnp.float32),
        pltpu.VMEM_SHARED((NP, 48), jnp.float32),
        pltpu.SemaphoreType.DMA,
        pltpu.SemaphoreType.DMA,
    ]

    @functools.partial(
        pl.kernel, mesh=_MESH,
        out_type=jax.ShapeDtypeStruct((2, NP, 48), jnp.float32),
        scratch_types=scratch, compiler_params=_NOTC)
    def l3_kernel(srcs_hbm, dsts_hbm, zeros_hbm, table, out_hbm,
                  sidx, didx, rows0, rows1, acc, sem0, sem1):
        c = lax.axis_index("c")
        s = lax.axis_index("s")
        w = c * 16 + s
        pltpu.sync_copy(srcs_hbm.at[w], sidx)
        pltpu.sync_copy(dsts_hbm.at[w], didx)
        pltpu.sync_copy(zeros_hbm.at[pl.ds(s * RPT, RPT)],
                        acc.at[pl.ds(s * RPT, RPT)])
        _agg_body(CHE, table, out_hbm, sidx, didx, rows0, rows1, acc,
                  sem0, sem1, c, s)

    return l3_kernel


def _dot(a, b):
    return lax.dot_general(a, b, (((1,), (0,)), ((), ())),
                           precision=lax.Precision.HIGHEST,
                           preferred_element_type=jnp.float32)


def _row_spec(width):
    return pl.BlockSpec((BR, width), lambda i: (i, 0))


def _half_spec():
    return pl.BlockSpec((2, BR, 64), lambda i: (0, i, 0))


def _part_spec(width):
    return pl.BlockSpec((2, BR, width), lambda i: (0, i, 0))


def _full_spec(shape):
    ndim = len(shape)
    return pl.BlockSpec(shape, lambda i, _n=ndim: (0,) * _n)


def _split2(h):
    return jnp.stack([h[:, :64], h[:, 64:128]], axis=0)


def _tc0_body(pref, xref, dinv_out, xp_out):
    deg = pref[0] + pref[1] + 1.0
    dinv = lax.rsqrt(deg)
    dinv_out[...] = dinv
    xp_out[...] = _split2(xref[...] * dinv[:, 0:1])


def _tc1_body(pref, xpref, dref, w1ref, b1ref, ta_out, tb_out):
    dinv = dref[...][:, 0:1]
    agg = jnp.concatenate([pref[0] + xpref[0], pref[1] + xpref[1]], axis=1)
    y = agg * dinv
    h = jnp.maximum(_dot(y, w1ref[...]) + b1ref[...], 0.0)
    hp = h * dinv
    ta_out[...] = _split2(hp[:, :128])
    tb_out[...] = _split2(hp[:, 128:])


def _tc2_body(paref, pbref, taref, tbref, dref, w2ref, b2ref, w3ref, h3_out):
    dinv = dref[...][:, 0:1]
    agg = jnp.concatenate([paref[0] + taref[0], paref[1] + taref[1],
                           pbref[0] + tbref[0], pbref[1] + tbref[1]], axis=1)
    y = agg * dinv
    h2 = jnp.maximum(_dot(y, w2ref[...]) + b2ref[...], 0.0)
    h3_out[...] = _dot(h2, w3ref[...]) * dinv


def _tc3_body(pref, h3ref, dref, b3ref, out):
    dinv = dref[...][:, 0:1]
    y = (pref[0] + pref[1] + h3ref[...]) * dinv + b3ref[...]
    mask = lax.broadcasted_iota(jnp.int32, (BR, 48), 1) < 40
    ym = jnp.where(mask, y, -jnp.inf)
    mx = jnp.max(ym, axis=1, keepdims=True)
    e = jnp.where(mask, jnp.exp(y - mx), 0.0)
    out[...] = y - mx - jnp.log(jnp.sum(e, axis=1, keepdims=True))


def kernel(x, edge_index, W1, b1, W2, b2, W3, b3):
    f32 = jnp.float32
    src = edge_index[0].astype(jnp.int32)
    dst = edge_index[1].astype(jnp.int32)
    pad_idx = N + (jnp.arange(EP - E, dtype=jnp.int32) % (NP - N))
    src_p = jnp.concatenate([src, pad_idx])
    dst_p = jnp.concatenate([dst, pad_idx])
    srcs_e = src_p.reshape(NW, CHE, C)
    dsts_e = dst_p.reshape(NW, CHE, C)
    src_c = src_p.reshape(16, CHC, C)
    srcs_c = jnp.concatenate([src_c, src_c + NP], axis=0)
    dsts_c = dst_p.reshape(16, CHC, C)

    x_pad = jnp.pad(x.astype(f32), ((0, NP - N), (0, 0)))
    ones16 = jnp.ones((C, 16), f32)
    zeros16 = jnp.zeros((NP, 16), f32)
    zeros48 = jnp.zeros((NP, 48), f32)
    zeros64 = jnp.zeros((NP, 64), f32)
    w3p = jnp.pad(W3.astype(f32), ((0, 0), (0, 8)))
    b1r = b1.astype(f32).reshape(1, 256)
    b2r = b2.astype(f32).reshape(1, 256)
    b3r = jnp.pad(b3.astype(f32), (0, 8)).reshape(1, 48)

    col_agg = _make_col_agg_kernel()

    deg_parts = _make_deg_kernel()(dsts_e, ones16, zeros16)

    dinv, xp2 = pl.pallas_call(
        _tc0_body,
        grid=(G,),
        in_specs=[_part_spec(16), _row_spec(128)],
        out_specs=[_row_spec(16), _half_spec()],
        out_shape=[jax.ShapeDtypeStruct((NP, 16), f32),
                   jax.ShapeDtypeStruct((2, NP, 64), f32)],
    )(deg_parts, x_pad)

    p1 = col_agg(srcs_c, dsts_c, zeros64, xp2.reshape(2 * NP, 64))

    ta, tb = pl.pallas_call(
        _tc1_body,
        grid=(G,),
        in_specs=[_half_spec(), _half_spec(), _row_spec(16),
                  _full_spec((128, 256)), _full_spec((1, 256))],
        out_specs=[_half_spec(), _half_spec()],
        out_shape=[jax.ShapeDtypeStruct((2, NP, 64), f32),
                   jax.ShapeDtypeStruct((2, NP, 64), f32)],
    )(p1, xp2, dinv, W1.astype(f32), b1r)

    pa = col_agg(srcs_c, dsts_c, zeros64, ta.reshape(2 * NP, 64))
    pb = col_agg(srcs_c, dsts_c, zeros64, tb.reshape(2 * NP, 64))

    (h3,) = pl.pallas_call(
        _tc2_body,
        grid=(G,),
        in_specs=[_half_spec(), _half_spec(), _half_spec(), _half_spec(),
                  _row_spec(16), _full_spec((256, 256)),
                  _full_spec((1, 256)), _full_spec((256, 48))],
        out_specs=[_row_spec(48)],
        out_shape=[jax.ShapeDtypeStruct((NP, 48), f32)],
    )(pa, pb, ta, tb, dinv, W2.astype(f32), b2r, w3p)

    p3 = _make_l3_kernel()(srcs_e, dsts_e, zeros48, h3)

    (out48,) = pl.pallas_call(
        _tc3_body,
        grid=(G,),
        in_specs=[_part_spec(48), _row_spec(48), _row_spec(16),
                  _full_spec((1, 48))],
        out_specs=[_row_spec(48)],
        out_shape=[jax.ShapeDtypeStruct((NP, 48), f32)],
    )(p3, h3, dinv, b3r)

    return out48[:N, :40]


# speedup vs baseline: 26.6779x; 1.0770x over previous
import functools

import jax
import jax.numpy as jnp
from jax import lax
from jax.experimental import pallas as pl
from jax.experimental.pallas import tpu as pltpu
from jax.experimental.pallas import tpu_sc as plsc

N = 10000
NP = 10240
E = 320000
NW = 32
C = 256
EP = 327680
CHE = EP // (NW * C)
CHC = EP // (16 * C)
RPT = NP // 16
BR = 1024
G = NP // BR

_MESH = plsc.VectorSubcoreMesh(core_axis_name="c", subcore_axis_name="s")
_NOTC = pltpu.CompilerParams(use_tc_tiling_on_sc=False)


def _make_deg_kernel():
    scratch = [
        pltpu.VMEM((CHE, C), jnp.int32),
        pltpu.VMEM((C, 16), jnp.float32),
        pltpu.VMEM_SHARED((NP, 16), jnp.float32),
    ]

    @functools.partial(
        pl.kernel, mesh=_MESH,
        out_type=jax.ShapeDtypeStruct((2, NP, 16), jnp.float32),
        scratch_types=scratch, compiler_params=_NOTC)
    def deg_kernel(dsts_hbm, ones_hbm, zeros_hbm, out_hbm, didx, ones_v, acc):
        c = lax.axis_index("c")
        s = lax.axis_index("s")
        w = c * 16 + s
        pltpu.sync_copy(dsts_hbm.at[w], didx)
        pltpu.sync_copy(ones_hbm, ones_v)
        pltpu.sync_copy(zeros_hbm.at[pl.ds(s * RPT, RPT)],
                        acc.at[pl.ds(s * RPT, RPT)])
        plsc.subcore_barrier()
        for j in range(CHE):
            pltpu.sync_copy(ones_v, acc.at[didx.at[j]], add=True)
        plsc.subcore_barrier()
        pltpu.sync_copy(acc.at[pl.ds(s * RPT, RPT)],
                        out_hbm.at[c].at[pl.ds(s * RPT, RPT)])

    return deg_kernel


def _agg_body(nch, table, out_hbm, sidx, didx, bufs, sems, acc, c, s):
    nb = len(bufs)
    plsc.subcore_barrier()
    cps = [None] * nb
    for b in range(nb - 1):
        cps[b] = pltpu.async_copy(table.at[sidx.at[b]], bufs[b], sems[b])
    for j in range(nch):
        g = j + nb - 1
        if g < nch:
            cps[g % nb] = pltpu.async_copy(
                table.at[sidx.at[g]], bufs[g % nb], sems[g % nb])
        cps[j % nb].wait()
        pltpu.sync_copy(bufs[j % nb], acc.at[didx.at[j]], add=True)
    plsc.subcore_barrier()
    pltpu.sync_copy(acc.at[pl.ds(s * RPT, RPT)],
                    out_hbm.at[c].at[pl.ds(s * RPT, RPT)])


def _make_col_agg_kernel(npass):
    scratch = [
        pltpu.VMEM((CHC, C), jnp.int32),
        pltpu.VMEM((CHC, C), jnp.int32),
        pltpu.VMEM((C, 64), jnp.float32),
        pltpu.VMEM((C, 64), jnp.float32),
        pltpu.VMEM((C, 64), jnp.float32),
        pltpu.VMEM_SHARED((NP, 64), jnp.float32),
        pltpu.SemaphoreType.DMA,
        pltpu.SemaphoreType.DMA,
        pltpu.SemaphoreType.DMA,
    ]

    @functools.partial(
        pl.kernel, mesh=_MESH,
        out_type=[jax.ShapeDtypeStruct((2, NP, 64), jnp.float32)] * npass,
        scratch_types=scratch, compiler_params=_NOTC)
    def col_kernel(srcs_hbm, dsts_hbm, zeros_hbm, *rest):
        tables = rest[:npass]
        outs = rest[npass:2 * npass]
        sidx, didx, r0, r1, r2, acc, s0, s1, s2 = rest[2 * npass:]
        c = lax.axis_index("c")
        s = lax.axis_index("s")
        pltpu.sync_copy(srcs_hbm.at[c * 16 + s], sidx)
        pltpu.sync_copy(dsts_hbm.at[s], didx)
        for p in range(npass):
            pltpu.sync_copy(zeros_hbm.at[pl.ds(s * RPT, RPT)],
                            acc.at[pl.ds(s * RPT, RPT)])
            _agg_body(CHC, tables[p], outs[p], sidx, didx, (r0, r1, r2),
                      (s0, s1, s2), acc, c, s)

    return col_kernel


def _make_l3_kernel():
    scratch = [
        pltpu.VMEM((CHE, C), jnp.int32),
        pltpu.VMEM((CHE, C), jnp.int32),
        pltpu.VMEM((C, 48), jnp.float32),
        pltpu.VMEM((C, 48), jnp.float32),
        pltpu.VMEM((C, 48), jnp.float32),
        pltpu.VMEM_SHARED((NP, 48), jnp.float32),
        pltpu.SemaphoreType.DMA,
        pltpu.SemaphoreType.DMA,
        pltpu.SemaphoreType.DMA,
    ]

    @functools.partial(
        pl.kernel, mesh=_MESH,
        out_type=jax.ShapeDtypeStruct((2, NP, 48), jnp.float32),
        scratch_types=scratch, compiler_params=_NOTC)
    def l3_kernel(srcs_hbm, dsts_hbm, zeros_hbm, table, out_hbm,
                  sidx, didx, r0, r1, r2, acc, s0, s1, s2):
        c = lax.axis_index("c")
        s = lax.axis_index("s")
        w = c * 16 + s
        pltpu.sync_copy(srcs_hbm.at[w], sidx)
        pltpu.sync_copy(dsts_hbm.at[w], didx)
        pltpu.sync_copy(zeros_hbm.at[pl.ds(s * RPT, RPT)],
                        acc.at[pl.ds(s * RPT, RPT)])
        _agg_body(CHE, table, out_hbm, sidx, didx, (r0, r1, r2),
                  (s0, s1, s2), acc, c, s)

    return l3_kernel


def _dot(a, b):
    return lax.dot_general(a, b, (((1,), (0,)), ((), ())),
                           precision=lax.Precision.HIGHEST,
                           preferred_element_type=jnp.float32)


def _row_spec(width):
    return pl.BlockSpec((BR, width), lambda i: (i, 0))


def _half_spec():
    return pl.BlockSpec((2, BR, 64), lambda i: (0, i, 0))


def _part_spec(width):
    return pl.BlockSpec((2, BR, width), lambda i: (0, i, 0))


def _full_spec(shape):
    ndim = len(shape)
    return pl.BlockSpec(shape, lambda i, _n=ndim: (0,) * _n)


def _split2(h):
    return jnp.stack([h[:, :64], h[:, 64:128]], axis=0)


def _tc0_body(pref, xref, dinv_out, xp_out):
    deg = pref[0] + pref[1] + 1.0
    dinv = lax.rsqrt(deg)
    dinv_out[...] = dinv
    xp_out[...] = _split2(xref[...] * dinv[:, 0:1])


def _tc1_body(pref, xpref, dref, w1ref, b1ref, ta_out, tb_out):
    dinv = dref[...][:, 0:1]
    agg = jnp.concatenate([pref[0] + xpref[0], pref[1] + xpref[1]], axis=1)
    y = agg * dinv
    h = jnp.maximum(_dot(y, w1ref[...]) + b1ref[...], 0.0)
    hp = h * dinv
    ta_out[...] = _split2(hp[:, :128])
    tb_out[...] = _split2(hp[:, 128:])


def _tc2_body(paref, pbref, taref, tbref, dref, w2ref, b2ref, w3ref, h3_out):
    dinv = dref[...][:, 0:1]
    agg = jnp.concatenate([paref[0] + taref[0], paref[1] + taref[1],
                           pbref[0] + tbref[0], pbref[1] + tbref[1]], axis=1)
    y = agg * dinv
    h2 = jnp.maximum(_dot(y, w2ref[...]) + b2ref[...], 0.0)
    h3_out[...] = _dot(h2, w3ref[...]) * dinv


def _tc3_body(pref, h3ref, dref, b3ref, out):
    dinv = dref[...][:, 0:1]
    y = (pref[0] + pref[1] + h3ref[...]) * dinv + b3ref[...]
    mask = lax.broadcasted_iota(jnp.int32, (BR, 48), 1) < 40
    ym = jnp.where(mask, y, -jnp.inf)
    mx = jnp.max(ym, axis=1, keepdims=True)
    e = jnp.where(mask, jnp.exp(y - mx), 0.0)
    out[...] = y - mx - jnp.log(jnp.sum(e, axis=1, keepdims=True))


def kernel(x, edge_index, W1, b1, W2, b2, W3, b3):
    f32 = jnp.float32
    src = edge_index[0].astype(jnp.int32)
    dst = edge_index[1].astype(jnp.int32)
    pad_idx = N + (jnp.arange(EP - E, dtype=jnp.int32) % (NP - N))
    src_p = jnp.concatenate([src, pad_idx])
    dst_p = jnp.concatenate([dst, pad_idx])
    srcs_e = src_p.reshape(NW, CHE, C)
    dsts_e = dst_p.reshape(NW, CHE, C)
    src_c = src_p.reshape(16, CHC, C)
    srcs_c = jnp.concatenate([src_c, src_c + NP], axis=0)
    dsts_c = dst_p.reshape(16, CHC, C)

    x_pad = jnp.pad(x.astype(f32), ((0, NP - N), (0, 0)))
    ones16 = jnp.ones((C, 16), f32)
    zeros16 = jnp.zeros((NP, 16), f32)
    zeros48 = jnp.zeros((NP, 48), f32)
    zeros64 = jnp.zeros((NP, 64), f32)
    w3p = jnp.pad(W3.astype(f32), ((0, 0), (0, 8)))
    b1r = b1.astype(f32).reshape(1, 256)
    b2r = b2.astype(f32).reshape(1, 256)
    b3r = jnp.pad(b3.astype(f32), (0, 8)).reshape(1, 48)


    deg_parts = _make_deg_kernel()(dsts_e, ones16, zeros16)

    dinv, xp2 = pl.pallas_call(
        _tc0_body,
        grid=(G,),
        in_specs=[_part_spec(16), _row_spec(128)],
        out_specs=[_row_spec(16), _half_spec()],
        out_shape=[jax.ShapeDtypeStruct((NP, 16), f32),
                   jax.ShapeDtypeStruct((2, NP, 64), f32)],
    )(deg_parts, x_pad)

    p1 = _make_col_agg_kernel(1)(srcs_c, dsts_c, zeros64,
                                 xp2.reshape(2 * NP, 64))[0]

    ta, tb = pl.pallas_call(
        _tc1_body,
        grid=(G,),
        in_specs=[_half_spec(), _half_spec(), _row_spec(16),
                  _full_spec((128, 256)), _full_spec((1, 256))],
        out_specs=[_half_spec(), _half_spec()],
        out_shape=[jax.ShapeDtypeStruct((2, NP, 64), f32),
                   jax.ShapeDtypeStruct((2, NP, 64), f32)],
    )(p1, xp2, dinv, W1.astype(f32), b1r)

    pa, pb = _make_col_agg_kernel(2)(srcs_c, dsts_c, zeros64,
                                     ta.reshape(2 * NP, 64),
                                     tb.reshape(2 * NP, 64))

    (h3,) = pl.pallas_call(
        _tc2_body,
        grid=(G,),
        in_specs=[_half_spec(), _half_spec(), _half_spec(), _half_spec(),
                  _row_spec(16), _full_spec((256, 256)),
                  _full_spec((1, 256)), _full_spec((256, 48))],
        out_specs=[_row_spec(48)],
        out_shape=[jax.ShapeDtypeStruct((NP, 48), f32)],
    )(pa, pb, ta, tb, dinv, W2.astype(f32), b2r, w3p)

    p3 = _make_l3_kernel()(srcs_e, dsts_e, zeros48, h3)

    (out48,) = pl.pallas_call(
        _tc3_body,
        grid=(G,),
        in_specs=[_part_spec(48), _row_spec(48), _row_spec(16),
                  _full_spec((1, 48))],
        out_specs=[_row_spec(48)],
        out_shape=[jax.ShapeDtypeStruct((NP, 48), f32)],
    )(p3, h3, dinv, b3r)

    return out48[:N, :40]


# speedup vs baseline: 26.7320x; 1.0020x over previous
import functools

import jax
import jax.numpy as jnp
from jax import lax
from jax.experimental import pallas as pl
from jax.experimental.pallas import tpu as pltpu
from jax.experimental.pallas import tpu_sc as plsc

N = 10000
NP = 10240
E = 320000
NW = 32
C = 256
EP = 327680
CHE = EP // (NW * C)
CHC = EP // (16 * C)
RPT = NP // 16
BR = 1024
G = NP // BR

_MESH = plsc.VectorSubcoreMesh(core_axis_name="c", subcore_axis_name="s")
_NOTC = pltpu.CompilerParams(use_tc_tiling_on_sc=False)


def _make_deg_kernel():
    scratch = [
        pltpu.VMEM((CHE, C), jnp.int32),
        pltpu.VMEM((C, 16), jnp.float32),
        pltpu.VMEM_SHARED((NP, 16), jnp.float32),
    ]

    @functools.partial(
        pl.kernel, mesh=_MESH,
        out_type=jax.ShapeDtypeStruct((2, NP, 16), jnp.float32),
        scratch_types=scratch, compiler_params=_NOTC)
    def deg_kernel(dsts_hbm, ones_hbm, zeros_hbm, out_hbm, didx, ones_v, acc):
        c = lax.axis_index("c")
        s = lax.axis_index("s")
        w = c * 16 + s
        pltpu.sync_copy(dsts_hbm.at[w], didx)
        pltpu.sync_copy(ones_hbm, ones_v)
        pltpu.sync_copy(zeros_hbm.at[pl.ds(s * RPT, RPT)],
                        acc.at[pl.ds(s * RPT, RPT)])
        plsc.subcore_barrier()
        for j in range(CHE):
            pltpu.sync_copy(ones_v, acc.at[didx.at[j]], add=True)
        plsc.subcore_barrier()
        pltpu.sync_copy(acc.at[pl.ds(s * RPT, RPT)],
                        out_hbm.at[c].at[pl.ds(s * RPT, RPT)])

    return deg_kernel


def _agg_body(nch, table, out_hbm, sidx, didx, bufs, sems, ssems, acc, c, s):
    nb = len(bufs)
    plsc.subcore_barrier()
    cps = [None] * nb
    scps = [None] * nb

    def _drain_scatter(b):
        if scps[b] is not None:
            scps[b].wait()
            scps[b] = None

    for b in range(nb - 1):
        cps[b] = pltpu.async_copy(table.at[sidx.at[b]], bufs[b], sems[b])
    for j in range(nch):
        g = j + nb - 1
        if g < nch:
            b = g % nb
            _drain_scatter(b)
            cps[b] = pltpu.async_copy(table.at[sidx.at[g]], bufs[b], sems[b])
        cps[j % nb].wait()
        scps[j % nb] = pltpu.async_copy(bufs[j % nb], acc.at[didx.at[j]],
                                        ssems[j % nb], add=True)
    for b in range(nb):
        _drain_scatter(b)
    plsc.subcore_barrier()
    pltpu.sync_copy(acc.at[pl.ds(s * RPT, RPT)],
                    out_hbm.at[c].at[pl.ds(s * RPT, RPT)])


def _make_col_agg_kernel(npass):
    scratch = [
        pltpu.VMEM((CHC, C), jnp.int32),
        pltpu.VMEM((CHC, C), jnp.int32),
        pltpu.VMEM((C, 64), jnp.float32),
        pltpu.VMEM((C, 64), jnp.float32),
        pltpu.VMEM((C, 64), jnp.float32),
        pltpu.VMEM_SHARED((NP, 64), jnp.float32),
        pltpu.SemaphoreType.DMA,
        pltpu.SemaphoreType.DMA,
        pltpu.SemaphoreType.DMA,
        pltpu.SemaphoreType.DMA,
        pltpu.SemaphoreType.DMA,
        pltpu.SemaphoreType.DMA,
    ]

    @functools.partial(
        pl.kernel, mesh=_MESH,
        out_type=[jax.ShapeDtypeStruct((2, NP, 64), jnp.float32)] * npass,
        scratch_types=scratch, compiler_params=_NOTC)
    def col_kernel(srcs_hbm, dsts_hbm, zeros_hbm, *rest):
        tables = rest[:npass]
        outs = rest[npass:2 * npass]
        (sidx, didx, r0, r1, r2, acc,
         s0, s1, s2, t0, t1, t2) = rest[2 * npass:]
        c = lax.axis_index("c")
        s = lax.axis_index("s")
        pltpu.sync_copy(srcs_hbm.at[c * 16 + s], sidx)
        pltpu.sync_copy(dsts_hbm.at[s], didx)
        for p in range(npass):
            pltpu.sync_copy(zeros_hbm.at[pl.ds(s * RPT, RPT)],
                            acc.at[pl.ds(s * RPT, RPT)])
            _agg_body(CHC, tables[p], outs[p], sidx, didx, (r0, r1, r2),
                      (s0, s1, s2), (t0, t1, t2), acc, c, s)

    return col_kernel


def _make_l3_kernel():
    scratch = [
        pltpu.VMEM((CHE, C), jnp.int32),
        pltpu.VMEM((CHE, C), jnp.int32),
        pltpu.VMEM((C, 48), jnp.float32),
        pltpu.VMEM((C, 48), jnp.float32),
        pltpu.VMEM((C, 48), jnp.float32),
        pltpu.VMEM_SHARED((NP, 48), jnp.float32),
        pltpu.SemaphoreType.DMA,
        pltpu.SemaphoreType.DMA,
        pltpu.SemaphoreType.DMA,
        pltpu.SemaphoreType.DMA,
        pltpu.SemaphoreType.DMA,
        pltpu.SemaphoreType.DMA,
    ]

    @functools.partial(
        pl.kernel, mesh=_MESH,
        out_type=jax.ShapeDtypeStruct((2, NP, 48), jnp.float32),
        scratch_types=scratch, compiler_params=_NOTC)
    def l3_kernel(srcs_hbm, dsts_hbm, zeros_hbm, table, out_hbm,
                  sidx, didx, r0, r1, r2, acc, s0, s1, s2, t0, t1, t2):
        c = lax.axis_index("c")
        s = lax.axis_index("s")
        w = c * 16 + s
        pltpu.sync_copy(srcs_hbm.at[w], sidx)
        pltpu.sync_copy(dsts_hbm.at[w], didx)
        pltpu.sync_copy(zeros_hbm.at[pl.ds(s * RPT, RPT)],
                        acc.at[pl.ds(s * RPT, RPT)])
        _agg_body(CHE, table, out_hbm, sidx, didx, (r0, r1, r2),
                  (s0, s1, s2), (t0, t1, t2), acc, c, s)

    return l3_kernel


def _dot(a, b):
    return lax.dot_general(a, b, (((1,), (0,)), ((), ())),
                           precision=lax.Precision.HIGHEST,
                           preferred_element_type=jnp.float32)


def _row_spec(width):
    return pl.BlockSpec((BR, width), lambda i: (i, 0))


def _half_spec():
    return pl.BlockSpec((2, BR, 64), lambda i: (0, i, 0))


def _part_spec(width):
    return pl.BlockSpec((2, BR, width), lambda i: (0, i, 0))


def _full_spec(shape):
    ndim = len(shape)
    return pl.BlockSpec(shape, lambda i, _n=ndim: (0,) * _n)


def _split2(h):
    return jnp.stack([h[:, :64], h[:, 64:128]], axis=0)


def _tc0_body(pref, xref, dinv_out, xp_out):
    deg = pref[0] + pref[1] + 1.0
    dinv = lax.rsqrt(deg)
    dinv_out[...] = dinv
    xp_out[...] = _split2(xref[...] * dinv[:, 0:1])


def _tc1_body(pref, xpref, dref, w1ref, b1ref, ta_out, tb_out):
    dinv = dref[...][:, 0:1]
    agg = jnp.concatenate([pref[0] + xpref[0], pref[1] + xpref[1]], axis=1)
    y = agg * dinv
    h = jnp.maximum(_dot(y, w1ref[...]) + b1ref[...], 0.0)
    hp = h * dinv
    ta_out[...] = _split2(hp[:, :128])
    tb_out[...] = _split2(hp[:, 128:])


def _tc2_body(paref, pbref, taref, tbref, dref, w2ref, b2ref, w3ref, h3_out):
    dinv = dref[...][:, 0:1]
    agg = jnp.concatenate([paref[0] + taref[0], paref[1] + taref[1],
                           pbref[0] + tbref[0], pbref[1] + tbref[1]], axis=1)
    y = agg * dinv
    h2 = jnp.maximum(_dot(y, w2ref[...]) + b2ref[...], 0.0)
    h3_out[...] = _dot(h2, w3ref[...]) * dinv


def _tc3_body(pref, h3ref, dref, b3ref, out):
    dinv = dref[...][:, 0:1]
    y = (pref[0] + pref[1] + h3ref[...]) * dinv + b3ref[...]
    mask = lax.broadcasted_iota(jnp.int32, (BR, 48), 1) < 40
    ym = jnp.where(mask, y, -jnp.inf)
    mx = jnp.max(ym, axis=1, keepdims=True)
    e = jnp.where(mask, jnp.exp(y - mx), 0.0)
    out[...] = y - mx - jnp.log(jnp.sum(e, axis=1, keepdims=True))


def kernel(x, edge_index, W1, b1, W2, b2, W3, b3):
    f32 = jnp.float32
    src = edge_index[0].astype(jnp.int32)
    dst = edge_index[1].astype(jnp.int32)
    pad_idx = N + (jnp.arange(EP - E, dtype=jnp.int32) % (NP - N))
    src_p = jnp.concatenate([src, pad_idx])
    dst_p = jnp.concatenate([dst, pad_idx])
    srcs_e = src_p.reshape(NW, CHE, C)
    dsts_e = dst_p.reshape(NW, CHE, C)
    src_c = src_p.reshape(16, CHC, C)
    srcs_c = jnp.concatenate([src_c, src_c + NP], axis=0)
    dsts_c = dst_p.reshape(16, CHC, C)

    x_pad = jnp.pad(x.astype(f32), ((0, NP - N), (0, 0)))
    ones16 = jnp.ones((C, 16), f32)
    zeros16 = jnp.zeros((NP, 16), f32)
    zeros48 = jnp.zeros((NP, 48), f32)
    zeros64 = jnp.zeros((NP, 64), f32)
    w3p = jnp.pad(W3.astype(f32), ((0, 0), (0, 8)))
    b1r = b1.astype(f32).reshape(1, 256)
    b2r = b2.astype(f32).reshape(1, 256)
    b3r = jnp.pad(b3.astype(f32), (0, 8)).reshape(1, 48)


    deg_parts = _make_deg_kernel()(dsts_e, ones16, zeros16)

    dinv, xp2 = pl.pallas_call(
        _tc0_body,
        grid=(G,),
        in_specs=[_part_spec(16), _row_spec(128)],
        out_specs=[_row_spec(16), _half_spec()],
        out_shape=[jax.ShapeDtypeStruct((NP, 16), f32),
                   jax.ShapeDtypeStruct((2, NP, 64), f32)],
    )(deg_parts, x_pad)

    p1 = _make_col_agg_kernel(1)(srcs_c, dsts_c, zeros64,
                                 xp2.reshape(2 * NP, 64))[0]

    ta, tb = pl.pallas_call(
        _tc1_body,
        grid=(G,),
        in_specs=[_half_spec(), _half_spec(), _row_spec(16),
                  _full_spec((128, 256)), _full_spec((1, 256))],
        out_specs=[_half_spec(), _half_spec()],
        out_shape=[jax.ShapeDtypeStruct((2, NP, 64), f32),
                   jax.ShapeDtypeStruct((2, NP, 64), f32)],
    )(p1, xp2, dinv, W1.astype(f32), b1r)

    pa, pb = _make_col_agg_kernel(2)(srcs_c, dsts_c, zeros64,
                                     ta.reshape(2 * NP, 64),
                                     tb.reshape(2 * NP, 64))

    (h3,) = pl.pallas_call(
        _tc2_body,
        grid=(G,),
        in_specs=[_half_spec(), _half_spec(), _half_spec(), _half_spec(),
                  _row_spec(16), _full_spec((256, 256)),
                  _full_spec((1, 256)), _full_spec((256, 48))],
        out_specs=[_row_spec(48)],
        out_shape=[jax.ShapeDtypeStruct((NP, 48), f32)],
    )(pa, pb, ta, tb, dinv, W2.astype(f32), b2r, w3p)

    p3 = _make_l3_kernel()(srcs_e, dsts_e, zeros48, h3)

    (out48,) = pl.pallas_call(
        _tc3_body,
        grid=(G,),
        in_specs=[_part_spec(48), _row_spec(48), _row_spec(16),
                  _full_spec((1, 48))],
        out_specs=[_row_spec(48)],
        out_shape=[jax.ShapeDtypeStruct((NP, 48), f32)],
    )(p3, h3, dinv, b3r)

    return out48[:N, :40]


# speedup vs baseline: 27.8001x; 1.0400x over previous
import functools

import jax
import jax.numpy as jnp
from jax import lax
from jax.experimental import pallas as pl
from jax.experimental.pallas import tpu as pltpu
from jax.experimental.pallas import tpu_sc as plsc

N = 10000
NP = 10240
E = 320000
NW = 32
C = 256
EP = 327680
CHE = EP // (NW * C)
CHC = EP // (16 * C)
RPT = NP // 16
BR = 1024
G = NP // BR

_MESH = plsc.VectorSubcoreMesh(core_axis_name="c", subcore_axis_name="s")
_NOTC = pltpu.CompilerParams(use_tc_tiling_on_sc=False)


def _make_deg_kernel():
    scratch = [
        pltpu.VMEM((CHE, C), jnp.int32),
        pltpu.VMEM((C, 16), jnp.float32),
        pltpu.VMEM_SHARED((NP, 16), jnp.float32),
    ]

    @functools.partial(
        pl.kernel, mesh=_MESH,
        out_type=jax.ShapeDtypeStruct((2, NP, 16), jnp.float32),
        scratch_types=scratch, compiler_params=_NOTC)
    def deg_kernel(dsts_hbm, ones_hbm, zeros_hbm, out_hbm, didx, ones_v, acc):
        c = lax.axis_index("c")
        s = lax.axis_index("s")
        w = c * 16 + s
        pltpu.sync_copy(dsts_hbm.at[w], didx)
        pltpu.sync_copy(ones_hbm, ones_v)
        pltpu.sync_copy(zeros_hbm.at[pl.ds(s * RPT, RPT)],
                        acc.at[pl.ds(s * RPT, RPT)])
        plsc.subcore_barrier()
        for j in range(CHE):
            pltpu.sync_copy(ones_v, acc.at[didx.at[j]], add=True)
        plsc.subcore_barrier()
        pltpu.sync_copy(acc.at[pl.ds(s * RPT, RPT)],
                        out_hbm.at[c].at[pl.ds(s * RPT, RPT)])

    return deg_kernel


def _agg_body(nch, table, out_hbm, sidx, didx, bufs, sems, ssems, acc, c, s):
    nb = len(bufs)
    plsc.subcore_barrier()
    cps = [None] * nb
    scps = [None] * nb

    def _drain_scatter(b):
        if scps[b] is not None:
            scps[b].wait()
            scps[b] = None

    for b in range(nb - 1):
        cps[b] = pltpu.async_copy(table.at[sidx.at[b]], bufs[b], sems[b])
    for j in range(nch):
        g = j + nb - 1
        if g < nch:
            b = g % nb
            _drain_scatter(b)
            cps[b] = pltpu.async_copy(table.at[sidx.at[g]], bufs[b], sems[b])
        cps[j % nb].wait()
        scps[j % nb] = pltpu.async_copy(bufs[j % nb], acc.at[didx.at[j]],
                                        ssems[j % nb], add=True)
    for b in range(nb):
        _drain_scatter(b)
    plsc.subcore_barrier()
    pltpu.sync_copy(acc.at[pl.ds(s * RPT, RPT)],
                    out_hbm.at[c].at[pl.ds(s * RPT, RPT)])


def _make_col_agg_kernel(npass):
    scratch = [
        pltpu.VMEM((CHC, C), jnp.int32),
        pltpu.VMEM((CHC, C), jnp.int32),
        pltpu.VMEM((C, 64), jnp.float32),
        pltpu.VMEM((C, 64), jnp.float32),
        pltpu.VMEM((C, 64), jnp.float32),
        pltpu.VMEM_SHARED((NP, 64), jnp.float32),
        pltpu.SemaphoreType.DMA,
        pltpu.SemaphoreType.DMA,
        pltpu.SemaphoreType.DMA,
        pltpu.SemaphoreType.DMA,
        pltpu.SemaphoreType.DMA,
        pltpu.SemaphoreType.DMA,
    ]

    @functools.partial(
        pl.kernel, mesh=_MESH,
        out_type=[jax.ShapeDtypeStruct((2, NP, 64), jnp.float32)] * npass,
        scratch_types=scratch, compiler_params=_NOTC)
    def col_kernel(srcs_hbm, dsts_hbm, zeros_hbm, *rest):
        tables = rest[:npass]
        outs = rest[npass:2 * npass]
        (sidx, didx, r0, r1, r2, acc,
         s0, s1, s2, t0, t1, t2) = rest[2 * npass:]
        c = lax.axis_index("c")
        s = lax.axis_index("s")
        pltpu.sync_copy(srcs_hbm.at[c * 16 + s], sidx)
        pltpu.sync_copy(dsts_hbm.at[s], didx)
        for p in range(npass):
            pltpu.sync_copy(zeros_hbm.at[pl.ds(s * RPT, RPT)],
                            acc.at[pl.ds(s * RPT, RPT)])
            _agg_body(CHC, tables[p], outs[p], sidx, didx, (r0, r1, r2),
                      (s0, s1, s2), (t0, t1, t2), acc, c, s)

    return col_kernel


def _make_l3_kernel():
    scratch = [
        pltpu.VMEM((CHE, C), jnp.int32),
        pltpu.VMEM((CHE, C), jnp.int32),
        pltpu.VMEM((C, 48), jnp.float32),
        pltpu.VMEM((C, 48), jnp.float32),
        pltpu.VMEM((C, 48), jnp.float32),
        pltpu.VMEM_SHARED((NP, 48), jnp.float32),
        pltpu.SemaphoreType.DMA,
        pltpu.SemaphoreType.DMA,
        pltpu.SemaphoreType.DMA,
        pltpu.SemaphoreType.DMA,
        pltpu.SemaphoreType.DMA,
        pltpu.SemaphoreType.DMA,
    ]

    @functools.partial(
        pl.kernel, mesh=_MESH,
        out_type=jax.ShapeDtypeStruct((2, NP, 48), jnp.float32),
        scratch_types=scratch, compiler_params=_NOTC)
    def l3_kernel(srcs_hbm, dsts_hbm, zeros_hbm, table, out_hbm,
                  sidx, didx, r0, r1, r2, acc, s0, s1, s2, t0, t1, t2):
        c = lax.axis_index("c")
        s = lax.axis_index("s")
        w = c * 16 + s
        pltpu.sync_copy(srcs_hbm.at[w], sidx)
        pltpu.sync_copy(dsts_hbm.at[w], didx)
        pltpu.sync_copy(zeros_hbm.at[pl.ds(s * RPT, RPT)],
                        acc.at[pl.ds(s * RPT, RPT)])
        _agg_body(CHE, table, out_hbm, sidx, didx, (r0, r1, r2),
                  (s0, s1, s2), (t0, t1, t2), acc, c, s)

    return l3_kernel


def _dot(a, b):
    return lax.dot_general(a, b, (((1,), (0,)), ((), ())),
                           precision=lax.Precision.DEFAULT,
                           preferred_element_type=jnp.float32)


def _row_spec(width):
    return pl.BlockSpec((BR, width), lambda i: (i, 0))


def _half_spec():
    return pl.BlockSpec((2, BR, 64), lambda i: (0, i, 0))


def _part_spec(width):
    return pl.BlockSpec((2, BR, width), lambda i: (0, i, 0))


def _full_spec(shape):
    ndim = len(shape)
    return pl.BlockSpec(shape, lambda i, _n=ndim: (0,) * _n)


def _split2(h):
    return jnp.stack([h[:, :64], h[:, 64:128]], axis=0)


def _tc0_body(pref, xref, dinv_out, xp_out):
    deg = pref[0] + pref[1] + 1.0
    dinv = lax.rsqrt(deg)
    dinv_out[...] = dinv
    xp_out[...] = _split2(xref[...] * dinv[:, 0:1])


def _tc1_body(pref, xpref, dref, w1ref, b1ref, ta_out, tb_out):
    dinv = dref[...][:, 0:1]
    agg = jnp.concatenate([pref[0] + xpref[0], pref[1] + xpref[1]], axis=1)
    y = agg * dinv
    h = jnp.maximum(_dot(y, w1ref[...]) + b1ref[...], 0.0)
    hp = h * dinv
    ta_out[...] = _split2(hp[:, :128])
    tb_out[...] = _split2(hp[:, 128:])


def _tc2_body(paref, pbref, taref, tbref, dref, w2ref, b2ref, w3ref, h3_out):
    dinv = dref[...][:, 0:1]
    agg = jnp.concatenate([paref[0] + taref[0], paref[1] + taref[1],
                           pbref[0] + tbref[0], pbref[1] + tbref[1]], axis=1)
    y = agg * dinv
    h2 = jnp.maximum(_dot(y, w2ref[...]) + b2ref[...], 0.0)
    h3_out[...] = _dot(h2, w3ref[...]) * dinv


def _tc3_body(pref, h3ref, dref, b3ref, out):
    dinv = dref[...][:, 0:1]
    y = (pref[0] + pref[1] + h3ref[...]) * dinv + b3ref[...]
    mask = lax.broadcasted_iota(jnp.int32, (BR, 48), 1) < 40
    ym = jnp.where(mask, y, -jnp.inf)
    mx = jnp.max(ym, axis=1, keepdims=True)
    e = jnp.where(mask, jnp.exp(y - mx), 0.0)
    out[...] = y - mx - jnp.log(jnp.sum(e, axis=1, keepdims=True))


def kernel(x, edge_index, W1, b1, W2, b2, W3, b3):
    f32 = jnp.float32
    src = edge_index[0].astype(jnp.int32)
    dst = edge_index[1].astype(jnp.int32)
    pad_idx = N + (jnp.arange(EP - E, dtype=jnp.int32) % (NP - N))
    src_p = jnp.concatenate([src, pad_idx])
    dst_p = jnp.concatenate([dst, pad_idx])
    srcs_e = src_p.reshape(NW, CHE, C)
    dsts_e = dst_p.reshape(NW, CHE, C)
    src_c = src_p.reshape(16, CHC, C)
    srcs_c = jnp.concatenate([src_c, src_c + NP], axis=0)
    dsts_c = dst_p.reshape(16, CHC, C)

    x_pad = jnp.pad(x.astype(f32), ((0, NP - N), (0, 0)))
    ones16 = jnp.ones((C, 16), f32)
    zeros16 = jnp.zeros((NP, 16), f32)
    zeros48 = jnp.zeros((NP, 48), f32)
    zeros64 = jnp.zeros((NP, 64), f32)
    w3p = jnp.pad(W3.astype(f32), ((0, 0), (0, 8)))
    b1r = b1.astype(f32).reshape(1, 256)
    b2r = b2.astype(f32).reshape(1, 256)
    b3r = jnp.pad(b3.astype(f32), (0, 8)).reshape(1, 48)


    deg_parts = _make_deg_kernel()(dsts_e, ones16, zeros16)

    dinv, xp2 = pl.pallas_call(
        _tc0_body,
        grid=(G,),
        in_specs=[_part_spec(16), _row_spec(128)],
        out_specs=[_row_spec(16), _half_spec()],
        out_shape=[jax.ShapeDtypeStruct((NP, 16), f32),
                   jax.ShapeDtypeStruct((2, NP, 64), f32)],
    )(deg_parts, x_pad)

    p1 = _make_col_agg_kernel(1)(srcs_c, dsts_c, zeros64,
                                 xp2.reshape(2 * NP, 64))[0]

    ta, tb = pl.pallas_call(
        _tc1_body,
        grid=(G,),
        in_specs=[_half_spec(), _half_spec(), _row_spec(16),
                  _full_spec((128, 256)), _full_spec((1, 256))],
        out_specs=[_half_spec(), _half_spec()],
        out_shape=[jax.ShapeDtypeStruct((2, NP, 64), f32),
                   jax.ShapeDtypeStruct((2, NP, 64), f32)],
    )(p1, xp2, dinv, W1.astype(f32), b1r)

    pa, pb = _make_col_agg_kernel(2)(srcs_c, dsts_c, zeros64,
                                     ta.reshape(2 * NP, 64),
                                     tb.reshape(2 * NP, 64))

    (h3,) = pl.pallas_call(
        _tc2_body,
        grid=(G,),
        in_specs=[_half_spec(), _half_spec(), _half_spec(), _half_spec(),
                  _row_spec(16), _full_spec((256, 256)),
                  _full_spec((1, 256)), _full_spec((256, 48))],
        out_specs=[_row_spec(48)],
        out_shape=[jax.ShapeDtypeStruct((NP, 48), f32)],
    )(pa, pb, ta, tb, dinv, W2.astype(f32), b2r, w3p)

    p3 = _make_l3_kernel()(srcs_e, dsts_e, zeros48, h3)

    (out48,) = pl.pallas_call(
        _tc3_body,
        grid=(G,),
        in_specs=[_part_spec(48), _row_spec(48), _row_spec(16),
                  _full_spec((1, 48))],
        out_specs=[_row_spec(48)],
        out_shape=[jax.ShapeDtypeStruct((NP, 48), f32)],
    )(p3, h3, dinv, b3r)

    return out48[:N, :40]


# speedup vs baseline: 27.8330x; 1.0012x over previous
import functools

import jax
import jax.numpy as jnp
from jax import lax
from jax.experimental import pallas as pl
from jax.experimental.pallas import tpu as pltpu
from jax.experimental.pallas import tpu_sc as plsc

N = 10000
NP = 10240
E = 320000
NW = 32
C = 256
EP = 327680
CHE = EP // (NW * C)
CHC = EP // (16 * C)
RPT = NP // 16
BR = 1024
G = NP // BR
BR3 = 1000

_MESH = plsc.VectorSubcoreMesh(core_axis_name="c", subcore_axis_name="s")
_NOTC = pltpu.CompilerParams(use_tc_tiling_on_sc=False)


def _make_deg_kernel():
    scratch = [
        pltpu.VMEM((CHE, C), jnp.int32),
        pltpu.VMEM((C, 16), jnp.float32),
        pltpu.VMEM_SHARED((NP, 16), jnp.float32),
    ]

    @functools.partial(
        pl.kernel, mesh=_MESH,
        out_type=jax.ShapeDtypeStruct((2, NP, 16), jnp.float32),
        scratch_types=scratch, compiler_params=_NOTC)
    def deg_kernel(dsts_hbm, ones_hbm, zeros_hbm, out_hbm, didx, ones_v, acc):
        c = lax.axis_index("c")
        s = lax.axis_index("s")
        w = c * 16 + s
        pltpu.sync_copy(dsts_hbm.at[w], didx)
        pltpu.sync_copy(ones_hbm, ones_v)
        pltpu.sync_copy(zeros_hbm.at[pl.ds(s * RPT, RPT)],
                        acc.at[pl.ds(s * RPT, RPT)])
        plsc.subcore_barrier()
        for j in range(CHE):
            pltpu.sync_copy(ones_v, acc.at[didx.at[j]], add=True)
        plsc.subcore_barrier()
        pltpu.sync_copy(acc.at[pl.ds(s * RPT, RPT)],
                        out_hbm.at[c].at[pl.ds(s * RPT, RPT)])

    return deg_kernel


def _agg_body(nch, table, out_hbm, sidx, didx, bufs, sems, ssems, acc, c, s):
    nb = len(bufs)
    plsc.subcore_barrier()
    cps = [None] * nb
    scps = [None] * nb

    def _drain_scatter(b):
        if scps[b] is not None:
            scps[b].wait()
            scps[b] = None

    for b in range(nb - 1):
        cps[b] = pltpu.async_copy(table.at[sidx.at[b]], bufs[b], sems[b])
    for j in range(nch):
        g = j + nb - 1
        if g < nch:
            b = g % nb
            _drain_scatter(b)
            cps[b] = pltpu.async_copy(table.at[sidx.at[g]], bufs[b], sems[b])
        cps[j % nb].wait()
        scps[j % nb] = pltpu.async_copy(bufs[j % nb], acc.at[didx.at[j]],
                                        ssems[j % nb], add=True)
    for b in range(nb):
        _drain_scatter(b)
    plsc.subcore_barrier()
    pltpu.sync_copy(acc.at[pl.ds(s * RPT, RPT)],
                    out_hbm.at[c].at[pl.ds(s * RPT, RPT)])


def _make_col_agg_kernel(npass):
    scratch = [
        pltpu.VMEM((CHC, C), jnp.int32),
        pltpu.VMEM((CHC, C), jnp.int32),
        pltpu.VMEM((C, 64), jnp.float32),
        pltpu.VMEM((C, 64), jnp.float32),
        pltpu.VMEM((C, 64), jnp.float32),
        pltpu.VMEM_SHARED((NP, 64), jnp.float32),
        pltpu.SemaphoreType.DMA,
        pltpu.SemaphoreType.DMA,
        pltpu.SemaphoreType.DMA,
        pltpu.SemaphoreType.DMA,
        pltpu.SemaphoreType.DMA,
        pltpu.SemaphoreType.DMA,
    ]

    @functools.partial(
        pl.kernel, mesh=_MESH,
        out_type=[jax.ShapeDtypeStruct((2, NP, 64), jnp.float32)] * npass,
        scratch_types=scratch, compiler_params=_NOTC)
    def col_kernel(srcs_hbm, dsts_hbm, zeros_hbm, *rest):
        tables = rest[:npass]
        outs = rest[npass:2 * npass]
        (sidx, didx, r0, r1, r2, acc,
         s0, s1, s2, t0, t1, t2) = rest[2 * npass:]
        c = lax.axis_index("c")
        s = lax.axis_index("s")
        pltpu.sync_copy(srcs_hbm.at[c * 16 + s], sidx)
        pltpu.sync_copy(dsts_hbm.at[s], didx)
        for p in range(npass):
            pltpu.sync_copy(zeros_hbm.at[pl.ds(s * RPT, RPT)],
                            acc.at[pl.ds(s * RPT, RPT)])
            _agg_body(CHC, tables[p], outs[p], sidx, didx, (r0, r1, r2),
                      (s0, s1, s2), (t0, t1, t2), acc, c, s)

    return col_kernel


def _make_l3_kernel():
    scratch = [
        pltpu.VMEM((CHE, C), jnp.int32),
        pltpu.VMEM((CHE, C), jnp.int32),
        pltpu.VMEM((C, 48), jnp.float32),
        pltpu.VMEM((C, 48), jnp.float32),
        pltpu.VMEM((C, 48), jnp.float32),
        pltpu.VMEM_SHARED((NP, 48), jnp.float32),
        pltpu.SemaphoreType.DMA,
        pltpu.SemaphoreType.DMA,
        pltpu.SemaphoreType.DMA,
        pltpu.SemaphoreType.DMA,
        pltpu.SemaphoreType.DMA,
        pltpu.SemaphoreType.DMA,
    ]

    @functools.partial(
        pl.kernel, mesh=_MESH,
        out_type=jax.ShapeDtypeStruct((2, NP, 48), jnp.float32),
        scratch_types=scratch, compiler_params=_NOTC)
    def l3_kernel(srcs_hbm, dsts_hbm, zeros_hbm, table, out_hbm,
                  sidx, didx, r0, r1, r2, acc, s0, s1, s2, t0, t1, t2):
        c = lax.axis_index("c")
        s = lax.axis_index("s")
        w = c * 16 + s
        pltpu.sync_copy(srcs_hbm.at[w], sidx)
        pltpu.sync_copy(dsts_hbm.at[w], didx)
        pltpu.sync_copy(zeros_hbm.at[pl.ds(s * RPT, RPT)],
                        acc.at[pl.ds(s * RPT, RPT)])
        _agg_body(CHE, table, out_hbm, sidx, didx, (r0, r1, r2),
                  (s0, s1, s2), (t0, t1, t2), acc, c, s)

    return l3_kernel


def _dot(a, b):
    return lax.dot_general(a, b, (((1,), (0,)), ((), ())),
                           precision=lax.Precision.DEFAULT,
                           preferred_element_type=jnp.float32)


def _row_spec(width):
    return pl.BlockSpec((BR, width), lambda i: (i, 0))


def _half_spec():
    return pl.BlockSpec((2, BR, 64), lambda i: (0, i, 0))


def _part_spec(width):
    return pl.BlockSpec((2, BR, width), lambda i: (0, i, 0))


def _full_spec(shape):
    ndim = len(shape)
    return pl.BlockSpec(shape, lambda i, _n=ndim: (0,) * _n)


def _split2(h):
    return jnp.stack([h[:, :64], h[:, 64:128]], axis=0)


def _tc0_body(pref, xref, dinv_out, xp_out):
    deg = pref[0] + pref[1] + 1.0
    dinv = lax.rsqrt(deg)
    dinv_out[...] = dinv
    xp_out[...] = _split2(xref[...] * dinv[:, 0:1])


def _tc1_body(pref, xpref, dref, w1ref, b1ref, ta_out, tb_out):
    dinv = dref[...][:, 0:1]
    agg = jnp.concatenate([pref[0] + xpref[0], pref[1] + xpref[1]], axis=1)
    y = agg * dinv
    h = jnp.maximum(_dot(y, w1ref[...]) + b1ref[...], 0.0)
    hp = h * dinv
    ta_out[...] = _split2(hp[:, :128])
    tb_out[...] = _split2(hp[:, 128:])


def _tc2_body(paref, pbref, taref, tbref, dref, w2ref, b2ref, w3ref, h3_out):
    dinv = dref[...][:, 0:1]
    agg = jnp.concatenate([paref[0] + taref[0], paref[1] + taref[1],
                           pbref[0] + tbref[0], pbref[1] + tbref[1]], axis=1)
    y = agg * dinv
    h2 = jnp.maximum(_dot(y, w2ref[...]) + b2ref[...], 0.0)
    h3_out[...] = _dot(h2, w3ref[...]) * dinv


def _tc3_body(pref, h3ref, dref, b3ref, out):
    dinv = dref[...][:, 0:1]
    y = (pref[0] + pref[1] + h3ref[...]) * dinv + b3ref[...]
    mask = lax.broadcasted_iota(jnp.int32, (BR3, 48), 1) < 40
    ym = jnp.where(mask, y, -jnp.inf)
    mx = jnp.max(ym, axis=1, keepdims=True)
    e = jnp.where(mask, jnp.exp(y - mx), 0.0)
    out[...] = (y - mx - jnp.log(jnp.sum(e, axis=1, keepdims=True)))[:, :40]


def kernel(x, edge_index, W1, b1, W2, b2, W3, b3):
    f32 = jnp.float32
    src = edge_index[0].astype(jnp.int32)
    dst = edge_index[1].astype(jnp.int32)
    pad_idx = N + (jnp.arange(EP - E, dtype=jnp.int32) % (NP - N))
    src_p = jnp.concatenate([src, pad_idx])
    dst_p = jnp.concatenate([dst, pad_idx])
    srcs_e = src_p.reshape(NW, CHE, C)
    dsts_e = dst_p.reshape(NW, CHE, C)
    src_c = src_p.reshape(16, CHC, C)
    srcs_c = jnp.concatenate([src_c, src_c + NP], axis=0)
    dsts_c = dst_p.reshape(16, CHC, C)

    x_pad = jnp.pad(x.astype(f32), ((0, NP - N), (0, 0)))
    ones16 = jnp.ones((C, 16), f32)
    zeros16 = jnp.zeros((NP, 16), f32)
    zeros48 = jnp.zeros((NP, 48), f32)
    zeros64 = jnp.zeros((NP, 64), f32)
    w3p = jnp.pad(W3.astype(f32), ((0, 0), (0, 8)))
    b1r = b1.astype(f32).reshape(1, 256)
    b2r = b2.astype(f32).reshape(1, 256)
    b3r = jnp.pad(b3.astype(f32), (0, 8)).reshape(1, 48)


    deg_parts = _make_deg_kernel()(dsts_e, ones16, zeros16)

    dinv, xp2 = pl.pallas_call(
        _tc0_body,
        grid=(G,),
        in_specs=[_part_spec(16), _row_spec(128)],
        out_specs=[_row_spec(16), _half_spec()],
        out_shape=[jax.ShapeDtypeStruct((NP, 16), f32),
                   jax.ShapeDtypeStruct((2, NP, 64), f32)],
    )(deg_parts, x_pad)

    p1 = _make_col_agg_kernel(1)(srcs_c, dsts_c, zeros64,
                                 xp2.reshape(2 * NP, 64))[0]

    ta, tb = pl.pallas_call(
        _tc1_body,
        grid=(G,),
        in_specs=[_half_spec(), _half_spec(), _row_spec(16),
                  _full_spec((128, 256)), _full_spec((1, 256))],
        out_specs=[_half_spec(), _half_spec()],
        out_shape=[jax.ShapeDtypeStruct((2, NP, 64), f32),
                   jax.ShapeDtypeStruct((2, NP, 64), f32)],
    )(p1, xp2, dinv, W1.astype(f32), b1r)

    pa, pb = _make_col_agg_kernel(2)(srcs_c, dsts_c, zeros64,
                                     ta.reshape(2 * NP, 64),
                                     tb.reshape(2 * NP, 64))

    (h3,) = pl.pallas_call(
        _tc2_body,
        grid=(G,),
        in_specs=[_half_spec(), _half_spec(), _half_spec(), _half_spec(),
                  _row_spec(16), _full_spec((256, 256)),
                  _full_spec((1, 256)), _full_spec((256, 48))],
        out_specs=[_row_spec(48)],
        out_shape=[jax.ShapeDtypeStruct((NP, 48), f32)],
    )(pa, pb, ta, tb, dinv, W2.astype(f32), b2r, w3p)

    p3 = _make_l3_kernel()(srcs_e, dsts_e, zeros48, h3)

    (out,) = pl.pallas_call(
        _tc3_body,
        grid=(N // BR3,),
        in_specs=[pl.BlockSpec((2, BR3, 48), lambda i: (0, i, 0)),
                  pl.BlockSpec((BR3, 48), lambda i: (i, 0)),
                  pl.BlockSpec((BR3, 16), lambda i: (i, 0)),
                  _full_spec((1, 48))],
        out_specs=[pl.BlockSpec((BR3, 40), lambda i: (i, 0))],
        out_shape=[jax.ShapeDtypeStruct((N, 40), f32)],
    )(p3, h3, dinv, b3r)

    return out


# speedup vs baseline: 28.3659x; 1.0191x over previous
import functools

import jax
import jax.numpy as jnp
from jax import lax
from jax.experimental import pallas as pl
from jax.experimental.pallas import tpu as pltpu
from jax.experimental.pallas import tpu_sc as plsc

N = 10000
NP = 10240
E = 320000
NW = 32
C = 256
EP = 327680
CHE = EP // (NW * C)
CHC = EP // (16 * C)
RPT = NP // 16
BR = 1024
G = NP // BR
BR3 = 1000

_MESH = plsc.VectorSubcoreMesh(core_axis_name="c", subcore_axis_name="s")
_NOTC = pltpu.CompilerParams(use_tc_tiling_on_sc=False)


def _make_deg_kernel():
    scratch = [
        pltpu.VMEM((CHE, C), jnp.int32),
        pltpu.VMEM((C, 16), jnp.float32),
        pltpu.VMEM_SHARED((NP, 16), jnp.float32),
    ]

    @functools.partial(
        pl.kernel, mesh=_MESH,
        out_type=jax.ShapeDtypeStruct((2, NP, 16), jnp.float32),
        scratch_types=scratch, compiler_params=_NOTC)
    def deg_kernel(dsts_hbm, ones_hbm, zeros_hbm, out_hbm, didx, ones_v, acc):
        c = lax.axis_index("c")
        s = lax.axis_index("s")
        w = c * 16 + s
        pltpu.sync_copy(dsts_hbm.at[w], didx)
        pltpu.sync_copy(ones_hbm, ones_v)
        pltpu.sync_copy(zeros_hbm.at[pl.ds(s * RPT, RPT)],
                        acc.at[pl.ds(s * RPT, RPT)])
        plsc.subcore_barrier()
        for j in range(CHE):
            pltpu.sync_copy(ones_v, acc.at[didx.at[j]], add=True)
        plsc.subcore_barrier()
        pltpu.sync_copy(acc.at[pl.ds(s * RPT, RPT)],
                        out_hbm.at[c].at[pl.ds(s * RPT, RPT)])

    return deg_kernel


def _agg_prime(table, sidx, bufs, sems):
    return [pltpu.async_copy(table.at[sidx.at[b]], bufs[b], sems[b])
            for b in range(len(bufs) - 1)] + [None]


def _agg_main(nch, table, sidx, didx, bufs, sems, ssems, acc, cps):
    nb = len(bufs)
    scps = [None] * nb

    def _drain_scatter(b):
        if scps[b] is not None:
            scps[b].wait()
            scps[b] = None

    for j in range(nch):
        g = j + nb - 1
        if g < nch:
            b = g % nb
            _drain_scatter(b)
            cps[b] = pltpu.async_copy(table.at[sidx.at[g]], bufs[b], sems[b])
        cps[j % nb].wait()
        scps[j % nb] = pltpu.async_copy(bufs[j % nb], acc.at[didx.at[j]],
                                        ssems[j % nb], add=True)
    for b in range(nb):
        _drain_scatter(b)


def _make_col_agg_kernel(npass):
    scratch = [
        pltpu.VMEM((CHC, C), jnp.int32),
        pltpu.VMEM((CHC, C), jnp.int32),
        pltpu.VMEM((C, 64), jnp.float32),
        pltpu.VMEM((C, 64), jnp.float32),
        pltpu.VMEM((C, 64), jnp.float32),
        pltpu.VMEM_SHARED((NP, 64), jnp.float32),
        pltpu.SemaphoreType.DMA,
        pltpu.SemaphoreType.DMA,
        pltpu.SemaphoreType.DMA,
        pltpu.SemaphoreType.DMA,
        pltpu.SemaphoreType.DMA,
        pltpu.SemaphoreType.DMA,
    ]

    @functools.partial(
        pl.kernel, mesh=_MESH,
        out_type=[jax.ShapeDtypeStruct((2, NP, 64), jnp.float32)] * npass,
        scratch_types=scratch, compiler_params=_NOTC)
    def col_kernel(srcs_hbm, dsts_hbm, zeros_hbm, *rest):
        tables = rest[:npass]
        outs = rest[npass:2 * npass]
        (sidx, didx, r0, r1, r2, acc,
         s0, s1, s2, t0, t1, t2) = rest[2 * npass:]
        c = lax.axis_index("c")
        s = lax.axis_index("s")
        pltpu.sync_copy(srcs_hbm.at[c * 16 + s], sidx)
        pltpu.sync_copy(dsts_hbm.at[s], didx)
        rows = pl.ds(s * RPT, RPT)
        for p in range(npass):
            cps = _agg_prime(tables[p], sidx, (r0, r1, r2), (s0, s1, s2))
            if p > 0:
                pltpu.sync_copy(acc.at[rows], outs[p - 1].at[c].at[rows])
            pltpu.sync_copy(zeros_hbm.at[rows], acc.at[rows])
            plsc.subcore_barrier()
            _agg_main(CHC, tables[p], sidx, didx, (r0, r1, r2),
                      (s0, s1, s2), (t0, t1, t2), acc, cps)
        plsc.subcore_barrier()
        pltpu.sync_copy(acc.at[rows], outs[npass - 1].at[c].at[rows])

    return col_kernel


def _make_l3_kernel():
    scratch = [
        pltpu.VMEM((CHE, C), jnp.int32),
        pltpu.VMEM((CHE, C), jnp.int32),
        pltpu.VMEM((C, 48), jnp.float32),
        pltpu.VMEM((C, 48), jnp.float32),
        pltpu.VMEM((C, 48), jnp.float32),
        pltpu.VMEM_SHARED((NP, 48), jnp.float32),
        pltpu.SemaphoreType.DMA,
        pltpu.SemaphoreType.DMA,
        pltpu.SemaphoreType.DMA,
        pltpu.SemaphoreType.DMA,
        pltpu.SemaphoreType.DMA,
        pltpu.SemaphoreType.DMA,
    ]

    @functools.partial(
        pl.kernel, mesh=_MESH,
        out_type=jax.ShapeDtypeStruct((2, NP, 48), jnp.float32),
        scratch_types=scratch, compiler_params=_NOTC)
    def l3_kernel(srcs_hbm, dsts_hbm, zeros_hbm, table, out_hbm,
                  sidx, didx, r0, r1, r2, acc, s0, s1, s2, t0, t1, t2):
        c = lax.axis_index("c")
        s = lax.axis_index("s")
        w = c * 16 + s
        rows = pl.ds(s * RPT, RPT)
        pltpu.sync_copy(srcs_hbm.at[w], sidx)
        pltpu.sync_copy(dsts_hbm.at[w], didx)
        cps = _agg_prime(table, sidx, (r0, r1, r2), (s0, s1, s2))
        pltpu.sync_copy(zeros_hbm.at[rows], acc.at[rows])
        plsc.subcore_barrier()
        _agg_main(CHE, table, sidx, didx, (r0, r1, r2),
                  (s0, s1, s2), (t0, t1, t2), acc, cps)
        plsc.subcore_barrier()
        pltpu.sync_copy(acc.at[rows], out_hbm.at[c].at[rows])

    return l3_kernel


def _dot(a, b):
    return lax.dot_general(a, b, (((1,), (0,)), ((), ())),
                           precision=lax.Precision.DEFAULT,
                           preferred_element_type=jnp.float32)


def _row_spec(width):
    return pl.BlockSpec((BR, width), lambda i: (i, 0))


def _half_spec():
    return pl.BlockSpec((2, BR, 64), lambda i: (0, i, 0))


def _part_spec(width):
    return pl.BlockSpec((2, BR, width), lambda i: (0, i, 0))


def _full_spec(shape):
    ndim = len(shape)
    return pl.BlockSpec(shape, lambda i, _n=ndim: (0,) * _n)


def _split2(h):
    return jnp.stack([h[:, :64], h[:, 64:128]], axis=0)


def _tc0_body(pref, xref, dinv_out, xp_out):
    deg = pref[0] + pref[1] + 1.0
    dinv = lax.rsqrt(deg)
    dinv_out[...] = dinv
    xp_out[...] = _split2(xref[...] * dinv[:, 0:1])


def _tc1_body(pref, xpref, dref, w1ref, b1ref, ta_out, tb_out):
    dinv = dref[...][:, 0:1]
    agg = jnp.concatenate([pref[0] + xpref[0], pref[1] + xpref[1]], axis=1)
    y = agg * dinv
    h = jnp.maximum(_dot(y, w1ref[...]) + b1ref[...], 0.0)
    hp = h * dinv
    ta_out[...] = _split2(hp[:, :128])
    tb_out[...] = _split2(hp[:, 128:])


def _tc2_body(paref, pbref, taref, tbref, dref, w2ref, b2ref, w3ref, h3_out):
    dinv = dref[...][:, 0:1]
    agg = jnp.concatenate([paref[0] + taref[0], paref[1] + taref[1],
                           pbref[0] + tbref[0], pbref[1] + tbref[1]], axis=1)
    y = agg * dinv
    h2 = jnp.maximum(_dot(y, w2ref[...]) + b2ref[...], 0.0)
    h3_out[...] = _dot(h2, w3ref[...]) * dinv


def _tc3_body(pref, h3ref, dref, b3ref, out):
    dinv = dref[...][:, 0:1]
    y = (pref[0] + pref[1] + h3ref[...]) * dinv + b3ref[...]
    mask = lax.broadcasted_iota(jnp.int32, (BR3, 48), 1) < 40
    ym = jnp.where(mask, y, -jnp.inf)
    mx = jnp.max(ym, axis=1, keepdims=True)
    e = jnp.where(mask, jnp.exp(y - mx), 0.0)
    out[...] = (y - mx - jnp.log(jnp.sum(e, axis=1, keepdims=True)))[:, :40]


def kernel(x, edge_index, W1, b1, W2, b2, W3, b3):
    f32 = jnp.float32
    src = edge_index[0].astype(jnp.int32)
    dst = edge_index[1].astype(jnp.int32)
    pad_idx = N + (jnp.arange(EP - E, dtype=jnp.int32) % (NP - N))
    src_p = jnp.concatenate([src, pad_idx])
    dst_p = jnp.concatenate([dst, pad_idx])
    srcs_e = src_p.reshape(NW, CHE, C)
    dsts_e = dst_p.reshape(NW, CHE, C)
    src_c = src_p.reshape(16, CHC, C)
    srcs_c = jnp.concatenate([src_c, src_c + NP], axis=0)
    dsts_c = dst_p.reshape(16, CHC, C)

    x_pad = jnp.pad(x.astype(f32), ((0, NP - N), (0, 0)))
    ones16 = jnp.ones((C, 16), f32)
    zeros16 = jnp.zeros((NP, 16), f32)
    zeros48 = jnp.zeros((NP, 48), f32)
    zeros64 = jnp.zeros((NP, 64), f32)
    w3p = jnp.pad(W3.astype(f32), ((0, 0), (0, 8)))
    b1r = b1.astype(f32).reshape(1, 256)
    b2r = b2.astype(f32).reshape(1, 256)
    b3r = jnp.pad(b3.astype(f32), (0, 8)).reshape(1, 48)


    deg_parts = _make_deg_kernel()(dsts_e, ones16, zeros16)

    dinv, xp2 = pl.pallas_call(
        _tc0_body,
        grid=(G,),
        in_specs=[_part_spec(16), _row_spec(128)],
        out_specs=[_row_spec(16), _half_spec()],
        out_shape=[jax.ShapeDtypeStruct((NP, 16), f32),
                   jax.ShapeDtypeStruct((2, NP, 64), f32)],
    )(deg_parts, x_pad)

    p1 = _make_col_agg_kernel(1)(srcs_c, dsts_c, zeros64,
                                 xp2.reshape(2 * NP, 64))[0]

    ta, tb = pl.pallas_call(
        _tc1_body,
        grid=(G,),
        in_specs=[_half_spec(), _half_spec(), _row_spec(16),
                  _full_spec((128, 256)), _full_spec((1, 256))],
        out_specs=[_half_spec(), _half_spec()],
        out_shape=[jax.ShapeDtypeStruct((2, NP, 64), f32),
                   jax.ShapeDtypeStruct((2, NP, 64), f32)],
    )(p1, xp2, dinv, W1.astype(f32), b1r)

    pa, pb = _make_col_agg_kernel(2)(srcs_c, dsts_c, zeros64,
                                     ta.reshape(2 * NP, 64),
                                     tb.reshape(2 * NP, 64))

    (h3,) = pl.pallas_call(
        _tc2_body,
        grid=(G,),
        in_specs=[_half_spec(), _half_spec(), _half_spec(), _half_spec(),
                  _row_spec(16), _full_spec((256, 256)),
                  _full_spec((1, 256)), _full_spec((256, 48))],
        out_specs=[_row_spec(48)],
        out_shape=[jax.ShapeDtypeStruct((NP, 48), f32)],
    )(pa, pb, ta, tb, dinv, W2.astype(f32), b2r, w3p)

    p3 = _make_l3_kernel()(srcs_e, dsts_e, zeros48, h3)

    (out,) = pl.pallas_call(
        _tc3_body,
        grid=(N // BR3,),
        in_specs=[pl.BlockSpec((2, BR3, 48), lambda i: (0, i, 0)),
                  pl.BlockSpec((BR3, 48), lambda i: (i, 0)),
                  pl.BlockSpec((BR3, 16), lambda i: (i, 0)),
                  _full_spec((1, 48))],
        out_specs=[pl.BlockSpec((BR3, 40), lambda i: (i, 0))],
        out_shape=[jax.ShapeDtypeStruct((N, 40), f32)],
    )(p3, h3, dinv, b3r)

    return out


# speedup vs baseline: 28.4395x; 1.0026x over previous
import functools

import jax
import jax.numpy as jnp
from jax import lax
from jax.experimental import pallas as pl
from jax.experimental.pallas import tpu as pltpu
from jax.experimental.pallas import tpu_sc as plsc

N = 10000
NP = 10240
E = 320000
NW = 32
C = 256
EP = 327680
CHE = EP // (NW * C)
CHC = EP // (16 * C)
RPT = NP // 16
BR = 1024
G = NP // BR
BR3 = 1000

_MESH = plsc.VectorSubcoreMesh(core_axis_name="c", subcore_axis_name="s")
_NOTC = pltpu.CompilerParams(use_tc_tiling_on_sc=False)


def _make_deg_kernel():
    scratch = [
        pltpu.VMEM((CHE, C), jnp.int32),
        pltpu.VMEM((C, 16), jnp.float32),
        pltpu.VMEM_SHARED((NP, 16), jnp.float32),
        pltpu.SemaphoreType.DMA,
        pltpu.SemaphoreType.DMA,
        pltpu.SemaphoreType.DMA,
        pltpu.SemaphoreType.DMA,
    ]

    @functools.partial(
        pl.kernel, mesh=_MESH,
        out_type=jax.ShapeDtypeStruct((2, NP, 16), jnp.float32),
        scratch_types=scratch, compiler_params=_NOTC)
    def deg_kernel(dsts_hbm, ones_hbm, zeros_hbm, out_hbm, didx, ones_v, acc,
                   d0, d1, d2, d3):
        c = lax.axis_index("c")
        s = lax.axis_index("s")
        w = c * 16 + s
        rows = pl.ds(s * RPT, RPT)
        dsems = (d0, d1, d2, d3)
        pltpu.sync_copy(dsts_hbm.at[w], didx)
        pltpu.sync_copy(ones_hbm, ones_v)
        pltpu.sync_copy(zeros_hbm.at[rows], acc.at[rows])
        plsc.subcore_barrier()
        hs = [None] * 4
        for j in range(CHE):
            b = j % 4
            if hs[b] is not None:
                hs[b].wait()
            hs[b] = pltpu.async_copy(ones_v, acc.at[didx.at[j]], dsems[b],
                                     add=True)
        for b in range(4):
            if hs[b] is not None:
                hs[b].wait()
        plsc.subcore_barrier()
        pltpu.sync_copy(acc.at[rows], out_hbm.at[c].at[rows])

    return deg_kernel


def _agg_prime(table, sidx, bufs, sems):
    return [pltpu.async_copy(table.at[sidx.at[b]], bufs[b], sems[b])
            for b in range(len(bufs) - 1)] + [None]


def _agg_main(nch, table, sidx, didx, bufs, sems, ssems, acc, cps):
    nb = len(bufs)
    scps = [None] * nb

    def _drain_scatter(b):
        if scps[b] is not None:
            scps[b].wait()
            scps[b] = None

    for j in range(nch):
        g = j + nb - 1
        if g < nch:
            b = g % nb
            _drain_scatter(b)
            cps[b] = pltpu.async_copy(table.at[sidx.at[g]], bufs[b], sems[b])
        cps[j % nb].wait()
        scps[j % nb] = pltpu.async_copy(bufs[j % nb], acc.at[didx.at[j]],
                                        ssems[j % nb], add=True)
    for b in range(nb):
        _drain_scatter(b)


def _make_col_agg_kernel(npass):
    scratch = [
        pltpu.VMEM((CHC, C), jnp.int32),
        pltpu.VMEM((CHC, C), jnp.int32),
        pltpu.VMEM((C, 64), jnp.float32),
        pltpu.VMEM((C, 64), jnp.float32),
        pltpu.VMEM((C, 64), jnp.float32),
        pltpu.VMEM_SHARED((NP, 64), jnp.float32),
        pltpu.SemaphoreType.DMA,
        pltpu.SemaphoreType.DMA,
        pltpu.SemaphoreType.DMA,
        pltpu.SemaphoreType.DMA,
        pltpu.SemaphoreType.DMA,
        pltpu.SemaphoreType.DMA,
    ]

    @functools.partial(
        pl.kernel, mesh=_MESH,
        out_type=[jax.ShapeDtypeStruct((2, NP, 64), jnp.float32)] * npass,
        scratch_types=scratch, compiler_params=_NOTC)
    def col_kernel(srcs_hbm, dsts_hbm, zeros_hbm, *rest):
        tables = rest[:npass]
        outs = rest[npass:2 * npass]
        (sidx, didx, r0, r1, r2, acc,
         s0, s1, s2, t0, t1, t2) = rest[2 * npass:]
        c = lax.axis_index("c")
        s = lax.axis_index("s")
        pltpu.sync_copy(srcs_hbm.at[c * 16 + s], sidx)
        pltpu.sync_copy(dsts_hbm.at[s], didx)
        rows = pl.ds(s * RPT, RPT)
        for p in range(npass):
            cps = _agg_prime(tables[p], sidx, (r0, r1, r2), (s0, s1, s2))
            if p > 0:
                pltpu.sync_copy(acc.at[rows], outs[p - 1].at[c].at[rows])
            pltpu.sync_copy(zeros_hbm.at[rows], acc.at[rows])
            plsc.subcore_barrier()
            _agg_main(CHC, tables[p], sidx, didx, (r0, r1, r2),
                      (s0, s1, s2), (t0, t1, t2), acc, cps)
        plsc.subcore_barrier()
        pltpu.sync_copy(acc.at[rows], outs[npass - 1].at[c].at[rows])

    return col_kernel


def _make_l3_kernel():
    scratch = [
        pltpu.VMEM((CHE, C), jnp.int32),
        pltpu.VMEM((CHE, C), jnp.int32),
        pltpu.VMEM((C, 48), jnp.float32),
        pltpu.VMEM((C, 48), jnp.float32),
        pltpu.VMEM((C, 48), jnp.float32),
        pltpu.VMEM_SHARED((NP, 48), jnp.float32),
        pltpu.SemaphoreType.DMA,
        pltpu.SemaphoreType.DMA,
        pltpu.SemaphoreType.DMA,
        pltpu.SemaphoreType.DMA,
        pltpu.SemaphoreType.DMA,
        pltpu.SemaphoreType.DMA,
    ]

    @functools.partial(
        pl.kernel, mesh=_MESH,
        out_type=jax.ShapeDtypeStruct((2, NP, 48), jnp.float32),
        scratch_types=scratch, compiler_params=_NOTC)
    def l3_kernel(srcs_hbm, dsts_hbm, zeros_hbm, table, out_hbm,
                  sidx, didx, r0, r1, r2, acc, s0, s1, s2, t0, t1, t2):
        c = lax.axis_index("c")
        s = lax.axis_index("s")
        w = c * 16 + s
        rows = pl.ds(s * RPT, RPT)
        pltpu.sync_copy(srcs_hbm.at[w], sidx)
        pltpu.sync_copy(dsts_hbm.at[w], didx)
        cps = _agg_prime(table, sidx, (r0, r1, r2), (s0, s1, s2))
        pltpu.sync_copy(zeros_hbm.at[rows], acc.at[rows])
        plsc.subcore_barrier()
        _agg_main(CHE, table, sidx, didx, (r0, r1, r2),
                  (s0, s1, s2), (t0, t1, t2), acc, cps)
        plsc.subcore_barrier()
        pltpu.sync_copy(acc.at[rows], out_hbm.at[c].at[rows])

    return l3_kernel


def _dot(a, b):
    return lax.dot_general(a, b, (((1,), (0,)), ((), ())),
                           precision=lax.Precision.DEFAULT,
                           preferred_element_type=jnp.float32)


def _row_spec(width):
    return pl.BlockSpec((BR, width), lambda i: (i, 0))


def _half_spec():
    return pl.BlockSpec((2, BR, 64), lambda i: (0, i, 0))


def _part_spec(width):
    return pl.BlockSpec((2, BR, width), lambda i: (0, i, 0))


def _full_spec(shape):
    ndim = len(shape)
    return pl.BlockSpec(shape, lambda i, _n=ndim: (0,) * _n)


def _split2(h):
    return jnp.stack([h[:, :64], h[:, 64:128]], axis=0)


def _tc0_body(pref, xref, dinv_out, xp_out):
    deg = pref[0] + pref[1] + 1.0
    dinv = lax.rsqrt(deg)
    dinv_out[...] = dinv
    xp_out[...] = _split2(xref[...] * dinv[:, 0:1])


def _tc1_body(pref, xpref, dref, w1ref, b1ref, ta_out, tb_out):
    dinv = dref[...][:, 0:1]
    agg = jnp.concatenate([pref[0] + xpref[0], pref[1] + xpref[1]], axis=1)
    y = agg * dinv
    h = jnp.maximum(_dot(y, w1ref[...]) + b1ref[...], 0.0)
    hp = h * dinv
    ta_out[...] = _split2(hp[:, :128])
    tb_out[...] = _split2(hp[:, 128:])


def _tc2_body(paref, pbref, taref, tbref, dref, w2ref, b2ref, w3ref, h3_out):
    dinv = dref[...][:, 0:1]
    agg = jnp.concatenate([paref[0] + taref[0], paref[1] + taref[1],
                           pbref[0] + tbref[0], pbref[1] + tbref[1]], axis=1)
    y = agg * dinv
    h2 = jnp.maximum(_dot(y, w2ref[...]) + b2ref[...], 0.0)
    h3_out[...] = _dot(h2, w3ref[...]) * dinv


def _tc3_body(pref, h3ref, dref, b3ref, out):
    dinv = dref[...][:, 0:1]
    y = (pref[0] + pref[1] + h3ref[...]) * dinv + b3ref[...]
    mask = lax.broadcasted_iota(jnp.int32, (BR3, 48), 1) < 40
    ym = jnp.where(mask, y, -jnp.inf)
    mx = jnp.max(ym, axis=1, keepdims=True)
    e = jnp.where(mask, jnp.exp(y - mx), 0.0)
    out[...] = (y - mx - jnp.log(jnp.sum(e, axis=1, keepdims=True)))[:, :40]


def kernel(x, edge_index, W1, b1, W2, b2, W3, b3):
    f32 = jnp.float32
    src = edge_index[0].astype(jnp.int32)
    dst = edge_index[1].astype(jnp.int32)
    pad_idx = N + (jnp.arange(EP - E, dtype=jnp.int32) % (NP - N))
    src_p = jnp.concatenate([src, pad_idx])
    dst_p = jnp.concatenate([dst, pad_idx])
    srcs_e = src_p.reshape(NW, CHE, C)
    dsts_e = dst_p.reshape(NW, CHE, C)
    src_c = src_p.reshape(16, CHC, C)
    srcs_c = jnp.concatenate([src_c, src_c + NP], axis=0)
    dsts_c = dst_p.reshape(16, CHC, C)

    x_pad = jnp.pad(x.astype(f32), ((0, NP - N), (0, 0)))
    ones16 = jnp.ones((C, 16), f32)
    zeros16 = jnp.zeros((NP, 16), f32)
    zeros48 = jnp.zeros((NP, 48), f32)
    zeros64 = jnp.zeros((NP, 64), f32)
    w3p = jnp.pad(W3.astype(f32), ((0, 0), (0, 8)))
    b1r = b1.astype(f32).reshape(1, 256)
    b2r = b2.astype(f32).reshape(1, 256)
    b3r = jnp.pad(b3.astype(f32), (0, 8)).reshape(1, 48)


    deg_parts = _make_deg_kernel()(dsts_e, ones16, zeros16)

    dinv, xp2 = pl.pallas_call(
        _tc0_body,
        grid=(G,),
        in_specs=[_part_spec(16), _row_spec(128)],
        out_specs=[_row_spec(16), _half_spec()],
        out_shape=[jax.ShapeDtypeStruct((NP, 16), f32),
                   jax.ShapeDtypeStruct((2, NP, 64), f32)],
    )(deg_parts, x_pad)

    p1 = _make_col_agg_kernel(1)(srcs_c, dsts_c, zeros64,
                                 xp2.reshape(2 * NP, 64))[0]

    ta, tb = pl.pallas_call(
        _tc1_body,
        grid=(G,),
        in_specs=[_half_spec(), _half_spec(), _row_spec(16),
                  _full_spec((128, 256)), _full_spec((1, 256))],
        out_specs=[_half_spec(), _half_spec()],
        out_shape=[jax.ShapeDtypeStruct((2, NP, 64), f32),
                   jax.ShapeDtypeStruct((2, NP, 64), f32)],
    )(p1, xp2, dinv, W1.astype(f32), b1r)

    pa, pb = _make_col_agg_kernel(2)(srcs_c, dsts_c, zeros64,
                                     ta.reshape(2 * NP, 64),
                                     tb.reshape(2 * NP, 64))

    (h3,) = pl.pallas_call(
        _tc2_body,
        grid=(G,),
        in_specs=[_half_spec(), _half_spec(), _half_spec(), _half_spec(),
                  _row_spec(16), _full_spec((256, 256)),
                  _full_spec((1, 256)), _full_spec((256, 48))],
        out_specs=[_row_spec(48)],
        out_shape=[jax.ShapeDtypeStruct((NP, 48), f32)],
    )(pa, pb, ta, tb, dinv, W2.astype(f32), b2r, w3p)

    p3 = _make_l3_kernel()(srcs_e, dsts_e, zeros48, h3)

    (out,) = pl.pallas_call(
        _tc3_body,
        grid=(N // BR3,),
        in_specs=[pl.BlockSpec((2, BR3, 48), lambda i: (0, i, 0)),
                  pl.BlockSpec((BR3, 48), lambda i: (i, 0)),
                  pl.BlockSpec((BR3, 16), lambda i: (i, 0)),
                  _full_spec((1, 48))],
        out_specs=[pl.BlockSpec((BR3, 40), lambda i: (i, 0))],
        out_shape=[jax.ShapeDtypeStruct((N, 40), f32)],
    )(p3, h3, dinv, b3r)

    return out
